# baseline (speedup 1.0000x reference)
import functools

import jax
import jax.numpy as jnp
from jax import lax
from jax.experimental import pallas as pl
from jax.experimental.pallas import tpu as pltpu
from jax.experimental.pallas import tpu_sc as plsc

_N = 10000
_E = 160000
_H = 128
_L = 4

_NC = 2
_NS = 16
_W = _NC * _NS

_C = 128
_EP = 163840
_NCH = _EP // _C
_CPW = _NCH // _W
_HCPW = _CPW // 2
_CPS = _NCH // _NS
_HCPS = _CPS // 2
_RPS = 640
_NP = _RPS * _NS
_DUMP = _N

_BE = 1024
_BN = 1000
_GE = _EP // _BE
_GN = _N // _BN


def _f32(shape):
    return jax.ShapeDtypeStruct(shape, jnp.float32)


def _sc_mesh():
    return plsc.VectorSubcoreMesh(
        core_axis_name="c", subcore_axis_name="s", num_cores=_NC,
        num_subcores=_NS)


@functools.cache
def _gather_rows(width):
    @functools.partial(
        pl.kernel,
        out_type=_f32((_EP, width)),
        mesh=_sc_mesh(),
        scratch_types=[
            pltpu.VMEM((_CPW, _C), jnp.int32),
            pltpu.VMEM((_C, width), jnp.float32),
            pltpu.VMEM((_C, width), jnp.float32),
            pltpu.SemaphoreType.DMA,
            pltpu.SemaphoreType.DMA,
            pltpu.SemaphoreType.DMA,
            pltpu.SemaphoreType.DMA,
        ],
    )
    def k(idx2, tab_h, out_o, idxv, b0, b1, g0, g1, st0, st1):
        wid = lax.axis_index("s") * _NC + lax.axis_index("c")
        cb = wid * _CPW

        pltpu.sync_copy(idx2.at[pl.ds(cb, _CPW)], idxv)

        def g_issue(jj, sa, sem):
            pltpu.async_copy(tab_h.at[idxv.at[jj]], sa, sem)

        def g_wait(jj, sa, sem):
            pltpu.make_async_copy(tab_h.at[idxv.at[jj]], sa, sem).wait()

        def s_issue(jj, sa, sem):
            pltpu.async_copy(sa, out_o.at[pl.ds((cb + jj) * _C, _C)], sem)

        def s_wait(jj, sa, sem):
            pltpu.make_async_copy(
                sa, out_o.at[pl.ds((cb + jj) * _C, _C)], sem).wait()

        g_issue(0, b0, g0)

        def body(j, carry):
            j0 = 2 * j
            j1 = 2 * j + 1

            @pl.when(j > 0)
            def _():
                s_wait(j0 - 1, b1, st1)

            g_issue(j1, b1, g1)
            g_wait(j0, b0, g0)
            s_issue(j0, b0, st0)

            @pl.when(j < _HCPW - 1)
            def _():
                s_wait(j0, b0, st0)
                g_issue(j0 + 2, b0, g0)

            g_wait(j1, b1, g1)
            s_issue(j1, b1, st1)
            return carry

        lax.fori_loop(0, _HCPW, body, 0)
        s_wait(_CPW - 2, b0, st0)
        s_wait(_CPW - 1, b1, st1)

    return k


@functools.cache
def _scatter2():
    @functools.partial(
        pl.kernel,
        out_type=(_f32((_N, _H)), _f32((_N, _H))),
        mesh=_sc_mesh(),
        scratch_types=[
            pltpu.VMEM((_CPS, _C), jnp.int32),
            pltpu.VMEM((_C, _H), jnp.float32),
            pltpu.VMEM((_C, _H), jnp.float32),
            pltpu.VMEM_SHARED((_NP, _H), jnp.float32),
            pltpu.SemaphoreType.DMA,
            pltpu.SemaphoreType.DMA,
        ],
    )
    def k(msg_h, sig_h, dst2, zeros_h, num_o, den_o,
          idxv, v0, v1, accum, l0, l1):
        c = lax.axis_index("c")
        s = lax.axis_index("s")
        sb = s * _CPS

        pltpu.sync_copy(zeros_h, accum.at[pl.ds(s * _RPS, _RPS)])
        pltpu.sync_copy(dst2.at[pl.ds(sb, _CPS)], idxv)
        plsc.subcore_barrier()

        def run(val_h):
            def v_issue(jj, v, sem):
                base = (sb + jj) * _C
                pltpu.async_copy(val_h.at[pl.ds(base, _C)], v, sem)

            def v_wait(jj, v, sem):
                base = (sb + jj) * _C
                pltpu.make_async_copy(
                    val_h.at[pl.ds(base, _C)], v, sem).wait()

            v_issue(0, v0, l0)

            def body(j, carry):
                j0 = 2 * j
                j1 = 2 * j + 1
                v_issue(j1, v1, l1)
                v_wait(j0, v0, l0)
                pltpu.sync_copy(v0, accum.at[idxv.at[j0]], add=True)

                @pl.when(j < _HCPS - 1)
                def _():
                    v_issue(j0 + 2, v0, l0)

                v_wait(j1, v1, l1)
                pltpu.sync_copy(v1, accum.at[idxv.at[j1]], add=True)
                return carry

            lax.fori_loop(0, _HCPS, body, 0)

        @pl.when(c == 0)
        def _():
            run(msg_h)

        @pl.when(c == 1)
        def _():
            run(sig_h)

        plsc.subcore_barrier()

        last = _N - _RPS * (_NS - 1)

        @pl.when(jnp.logical_and(c == 0, s < _NS - 1))
        def _():
            pltpu.sync_copy(accum.at[pl.ds(s * _RPS, _RPS)],
                            num_o.at[pl.ds(s * _RPS, _RPS)])

        @pl.when(jnp.logical_and(c == 0, s == _NS - 1))
        def _():
            pltpu.sync_copy(accum.at[pl.ds((_NS - 1) * _RPS, last)],
                            num_o.at[pl.ds((_NS - 1) * _RPS, last)])

        @pl.when(jnp.logical_and(c == 1, s < _NS - 1))
        def _():
            pltpu.sync_copy(accum.at[pl.ds(s * _RPS, _RPS)],
                            den_o.at[pl.ds(s * _RPS, _RPS)])

        @pl.when(jnp.logical_and(c == 1, s == _NS - 1))
        def _():
            pltpu.sync_copy(accum.at[pl.ds((_NS - 1) * _RPS, last)],
                            den_o.at[pl.ds((_NS - 1) * _RPS, last)])

    return k


def _row_spec(b, w):
    return pl.BlockSpec((b, w), lambda i: (i, 0))


def _col_spec(b, w, col):
    return pl.BlockSpec((b, w), lambda i, _c=col: (i, _c))


def _full_spec(r, w):
    return pl.BlockSpec((r, w), lambda i: (0, 0))


def _h0_body(pe_r, w_r, b_r, out_r):
    out_r[...] = (jnp.dot(pe_r[...], w_r[...],
                          preferred_element_type=jnp.float32) + b_r[...])


@functools.cache
def _h0_call():
    return pl.pallas_call(
        _h0_body,
        grid=(_GN,),
        in_specs=[_row_spec(_BN, 10), _full_spec(10, _H), _full_spec(1, _H)],
        out_specs=_row_spec(_BN, _H),
        out_shape=_f32((_N, _H)),
    )


def _eh0_body(e_r, w1_r, b1_r, w2_r, b2_r, out_r):
    t = jnp.dot(e_r[...], w1_r[...], preferred_element_type=jnp.float32)
    t = jnp.maximum(t + b1_r[...], 0.0)
    out_r[...] = (jnp.dot(t, w2_r[...], preferred_element_type=jnp.float32)
                  + b2_r[...])


@functools.cache
def _eh0_call():
    return pl.pallas_call(
        _eh0_body,
        grid=(_GE,),
        in_specs=[_row_spec(_BE, 16), _full_spec(16, 16), _full_spec(1, 16),
                  _full_spec(16, _H), _full_spec(1, _H)],
        out_specs=_row_spec(_BE, _H),
        out_shape=_f32((_EP, _H)),
    )


def _node_mm_body(h_r, w_r, b_r, a_o, db_o, ew_o):
    r = jnp.dot(h_r[...], w_r[...], preferred_element_type=jnp.float32)
    r = r + b_r[...]
    a_o[...] = r[:, 0 * _H:1 * _H]
    db_o[...] = r[:, 1 * _H:3 * _H]
    ew_o[...] = r[:, 3 * _H:4 * _H]


@functools.cache
def _node_mm_call():
    return pl.pallas_call(
        _node_mm_body,
        grid=(_GN,),
        in_specs=[_row_spec(_BN, _H), _full_spec(_H, 4 * _H),
                  _full_spec(1, 4 * _H)],
        out_specs=(_row_spec(_BN, _H), _row_spec(_BN, 2 * _H),
                   _row_spec(_BN, _H)),
        out_shape=(_f32((_N, _H)), _f32((_N, 2 * _H)), _f32((_N, _H))),
    )


def _edge1_body(eh_r, cw_r, cb_r, dhs_r, ehd_r, enew_o, stats_o):
    i = pl.program_id(0)
    v = jnp.dot(eh_r[...], cw_r[...], preferred_element_type=jnp.float32)
    v = v + cb_r[...] + dhs_r[...] + ehd_r[...]
    enew_o[...] = v
    rows = lax.broadcasted_iota(jnp.int32, (_BE, _H), 0) + i * _BE
    vm = jnp.where(rows < _E, v, 0.0)
    st = jnp.concatenate(
        [jnp.sum(vm, axis=0, keepdims=True),
         jnp.sum(vm * vm, axis=0, keepdims=True),
         jnp.zeros((6, _H), jnp.float32)], axis=0)

    @pl.when(i == 0)
    def _():
        stats_o[...] = st

    @pl.when(i > 0)
    def _():
        stats_o[...] += st


@functools.cache
def _edge1_call():
    return pl.pallas_call(
        _edge1_body,
        grid=(_GE,),
        in_specs=[_row_spec(_BE, _H), _full_spec(_H, _H), _full_spec(1, _H),
                  _col_spec(_BE, _H, 0), _row_spec(_BE, _H)],
        out_specs=(_row_spec(_BE, _H), _full_spec(8, _H)),
        out_shape=(_f32((_EP, _H)), _f32((8, _H))),
    )


def _edge2_body(eh_r, enew_r, bhs_r, st_r, g_r, b_r, ehn_o, sig_o, msg_o):
    st = st_r[...]
    m = st[0:1, :] * (1.0 / _E)
    var = st[1:2, :] * (1.0 / _E) - m * m
    inv = lax.rsqrt(var + 1e-5)
    xb = g_r[...] * (enew_r[...] - m) * inv + b_r[...]
    ehn = eh_r[...] + jnp.maximum(xb, 0.0)
    sig = jax.nn.sigmoid(ehn)
    ehn_o[...] = ehn
    sig_o[...] = sig
    msg_o[...] = sig * bhs_r[...]


@functools.cache
def _edge2_call():
    return pl.pallas_call(
        _edge2_body,
        grid=(_GE,),
        in_specs=[_row_spec(_BE, _H), _row_spec(_BE, _H),
                  _col_spec(_BE, _H, 1),
                  _full_spec(8, _H), _full_spec(1, _H), _full_spec(1, _H)],
        out_specs=tuple(_row_spec(_BE, _H) for _ in range(3)),
        out_shape=tuple(_f32((_EP, _H)) for _ in range(3)),
    )


def _node1_body(ah_r, num_r, den_r, t_o, stats_o):
    i = pl.program_id(0)
    v = ah_r[...] + num_r[...] / (den_r[...] + 1e-6)
    t_o[...] = v
    st = jnp.concatenate(
        [jnp.sum(v, axis=0, keepdims=True),
         jnp.sum(v * v, axis=0, keepdims=True),
         jnp.zeros((6, _H), jnp.float32)], axis=0)

    @pl.when(i == 0)
    def _():
        stats_o[...] = st

    @pl.when(i > 0)
    def _():
        stats_o[...] += st


@functools.cache
def _node1_call():
    return pl.pallas_call(
        _node1_body,
        grid=(_GN,),
        in_specs=[_row_spec(_BN, _H)] * 3,
        out_specs=(_row_spec(_BN, _H), _full_spec(8, _H)),
        out_shape=(_f32((_N, _H)), _f32((8, _H))),
    )


def _node2_body(h_r, t_r, st_r, g_r, b_r, h_o):
    st = st_r[...]
    m = st[0:1, :] * (1.0 / _N)
    var = st[1:2, :] * (1.0 / _N) - m * m
    inv = lax.rsqrt(var + 1e-5)
    xb = g_r[...] * (t_r[...] - m) * inv + b_r[...]
    h_o[...] = h_r[...] + jnp.maximum(xb, 0.0)


@functools.cache
def _node2_call():
    return pl.pallas_call(
        _node2_body,
        grid=(_GN,),
        in_specs=[_row_spec(_BN, _H), _row_spec(_BN, _H), _full_spec(8, _H),
                  _full_spec(1, _H), _full_spec(1, _H)],
        out_specs=_row_spec(_BN, _H),
        out_shape=_f32((_N, _H)),
    )


def _final_body(hs_r, hd_r, eh_r, pa_r, pb_r, pc_r, p1b_r, p2_r, p2b_r,
                out_o):
    z = jnp.dot(hs_r[...], pa_r[...], preferred_element_type=jnp.float32)
    z = z + jnp.dot(hd_r[...], pb_r[...], preferred_element_type=jnp.float32)
    z = z + jnp.dot(eh_r[...], pc_r[...], preferred_element_type=jnp.float32)
    z = jnp.maximum(z + p1b_r[...], 0.0)
    out_o[...] = (jnp.dot(z, p2_r[...], preferred_element_type=jnp.float32)
                  + p2b_r[...])


@functools.cache
def _final_call():
    return pl.pallas_call(
        _final_body,
        grid=(_GE,),
        in_specs=[_row_spec(_BE, _H)] * 3
        + [_full_spec(_H, _H)] * 3
        + [_full_spec(1, _H), _full_spec(_H, 1), _full_spec(1, 1)],
        out_specs=_row_spec(_BE, 1),
        out_shape=_f32((_EP, 1)),
    )


def kernel(edge_index, x, e, pe, pe_W, pe_b, e1_W, e1_b, e2_W, e2_b,
           A_W, A_b, B_W, B_b, C_W, C_b, D_W, D_b, Ew_W, Ew_b,
           bn_h_g, bn_h_b, bn_e_g, bn_e_b, p1_W, p1_b, p2_W, p2_b):
    pad = _EP - _E
    src2 = jnp.concatenate(
        [edge_index[0], jnp.zeros((pad,), jnp.int32)]).reshape(_NCH, _C)
    dstg2 = jnp.concatenate(
        [edge_index[1], jnp.zeros((pad,), jnp.int32)]).reshape(_NCH, _C)
    dump = _DUMP + (jnp.arange(pad, dtype=jnp.int32) % (_NP - _N))
    dsts2 = jnp.concatenate([edge_index[1], dump]).reshape(_NCH, _C)
    e_p = jnp.concatenate([e, jnp.zeros((pad, 16), jnp.float32)], axis=0)

    h = _h0_call()(pe, pe_W, pe_b.reshape(1, _H))
    eh = _eh0_call()(e_p, e1_W, e1_b.reshape(1, 16), e2_W,
                     e2_b.reshape(1, _H))

    W4 = jnp.concatenate([A_W, D_W, B_W, Ew_W], axis=2)
    b4 = jnp.concatenate([A_b, D_b, B_b, Ew_b], axis=1)
    zeros_n = jnp.zeros((_RPS, _H), jnp.float32)

    for l in range(_L):
        Ah, DBh, Ewh = _node_mm_call()(h, W4[l], b4[l].reshape(1, 4 * _H))
        dbs = _gather_rows(2 * _H)(src2, DBh)
        ehd = _gather_rows(_H)(dstg2, Ewh)
        enew, est = _edge1_call()(eh, C_W[l], C_b[l].reshape(1, _H), dbs, ehd)
        eh, sig, msg = _edge2_call()(eh, enew, dbs, est,
                                     bn_e_g[l].reshape(1, _H),
                                     bn_e_b[l].reshape(1, _H))
        num, den = _scatter2()(msg, sig, dsts2, zeros_n)
        t, nst = _node1_call()(Ah, num, den)
        h = _node2_call()(h, t, nst, bn_h_g[l].reshape(1, _H),
                          bn_h_b[l].reshape(1, _H))

    hs = _gather_rows(_H)(src2, h)
    hd = _gather_rows(_H)(dstg2, h)
    scores = _final_call()(hs, hd, eh,
                           p1_W[0 * _H:1 * _H], p1_W[1 * _H:2 * _H],
                           p1_W[2 * _H:3 * _H], p1_b.reshape(1, _H),
                           p2_W, p2_b.reshape(1, 1))
    return scores[:_E]


# speedup vs baseline: 1.1473x; 1.1473x over previous
import functools

import jax
import jax.numpy as jnp
from jax import lax
from jax.experimental import pallas as pl
from jax.experimental.pallas import tpu as pltpu
from jax.experimental.pallas import tpu_sc as plsc

_N = 10000
_E = 160000
_H = 128
_L = 4

_NC = 2
_NS = 16
_W = _NC * _NS

_C = 128
_EP = 163840
_NCH = _EP // _C
_CPW = _NCH // _W
_HCPW = _CPW // 2
_CPS = _NCH // _NS
_HCPS = _CPS // 2
_RPS = 640
_NP = _RPS * _NS
_DUMP = _N

_BE = 1024
_BN = 1000
_GE = _EP // _BE
_GN = _N // _BN


def _f32(shape):
    return jax.ShapeDtypeStruct(shape, jnp.float32)


def _sc_mesh():
    return plsc.VectorSubcoreMesh(
        core_axis_name="c", subcore_axis_name="s", num_cores=_NC,
        num_subcores=_NS)


@functools.cache
def _gather3():
    @functools.partial(
        pl.kernel,
        out_type=(_f32((_EP, _H)), _f32((_EP, _H)), _f32((_EP, _H))),
        mesh=_sc_mesh(),
        scratch_types=[
            pltpu.VMEM((_CPW, _C), jnp.int32),
            pltpu.VMEM((_CPW, _C), jnp.int32),
            pltpu.VMEM((_C, _H), jnp.float32),
            pltpu.VMEM((_C, _H), jnp.float32),
            pltpu.VMEM((_C, _H), jnp.float32),
            pltpu.VMEM((_C, _H), jnp.float32),
            pltpu.VMEM((_C, _H), jnp.float32),
            pltpu.VMEM((_C, _H), jnp.float32),
            pltpu.SemaphoreType.DMA,
            pltpu.SemaphoreType.DMA,
            pltpu.SemaphoreType.DMA,
            pltpu.SemaphoreType.DMA,
        ],
    )
    def k(src2, dst2, d_h, b_h, e_h, dhs_o, bhs_o, ehd_o,
          srcv, dstv, d0, b0, e0, d1, b1, e1, g0, g1, st0, st1):
        wid = lax.axis_index("s") * _NC + lax.axis_index("c")
        cb = wid * _CPW

        pltpu.sync_copy(src2.at[pl.ds(cb, _CPW)], srcv)
        pltpu.sync_copy(dst2.at[pl.ds(cb, _CPW)], dstv)

        def g_issue(jj, sd, sb, se, sem):
            pltpu.async_copy(d_h.at[srcv.at[jj]], sd, sem)
            pltpu.async_copy(b_h.at[srcv.at[jj]], sb, sem)
            pltpu.async_copy(e_h.at[dstv.at[jj]], se, sem)

        def g_wait(jj, sd, sb, se, sem):
            pltpu.make_async_copy(d_h.at[srcv.at[jj]], sd, sem).wait()
            pltpu.make_async_copy(b_h.at[srcv.at[jj]], sb, sem).wait()
            pltpu.make_async_copy(e_h.at[dstv.at[jj]], se, sem).wait()

        def s_issue(jj, sd, sb, se, sem):
            base = (cb + jj) * _C
            pltpu.async_copy(sd, dhs_o.at[pl.ds(base, _C)], sem)
            pltpu.async_copy(sb, bhs_o.at[pl.ds(base, _C)], sem)
            pltpu.async_copy(se, ehd_o.at[pl.ds(base, _C)], sem)

        def s_wait(jj, sd, sb, se, sem):
            base = (cb + jj) * _C
            pltpu.make_async_copy(sd, dhs_o.at[pl.ds(base, _C)], sem).wait()
            pltpu.make_async_copy(sb, bhs_o.at[pl.ds(base, _C)], sem).wait()
            pltpu.make_async_copy(se, ehd_o.at[pl.ds(base, _C)], sem).wait()

        g_issue(0, d0, b0, e0, g0)

        def body(j, carry):
            j0 = 2 * j
            j1 = 2 * j + 1

            @pl.when(j > 0)
            def _():
                s_wait(j0 - 1, d1, b1, e1, st1)

            g_issue(j1, d1, b1, e1, g1)
            g_wait(j0, d0, b0, e0, g0)
            s_issue(j0, d0, b0, e0, st0)

            @pl.when(j < _HCPW - 1)
            def _():
                s_wait(j0, d0, b0, e0, st0)
                g_issue(j0 + 2, d0, b0, e0, g0)

            g_wait(j1, d1, b1, e1, g1)
            s_issue(j1, d1, b1, e1, st1)
            return carry

        lax.fori_loop(0, _HCPW, body, 0)
        s_wait(_CPW - 2, d0, b0, e0, st0)
        s_wait(_CPW - 1, d1, b1, e1, st1)

    return k


@functools.cache
def _gather2():
    @functools.partial(
        pl.kernel,
        out_type=(_f32((_EP, _H)), _f32((_EP, _H))),
        mesh=_sc_mesh(),
        scratch_types=[
            pltpu.VMEM((_CPW, _C), jnp.int32),
            pltpu.VMEM((_CPW, _C), jnp.int32),
            pltpu.VMEM((_C, _H), jnp.float32),
            pltpu.VMEM((_C, _H), jnp.float32),
            pltpu.VMEM((_C, _H), jnp.float32),
            pltpu.VMEM((_C, _H), jnp.float32),
            pltpu.SemaphoreType.DMA,
            pltpu.SemaphoreType.DMA,
            pltpu.SemaphoreType.DMA,
            pltpu.SemaphoreType.DMA,
        ],
    )
    def k(src2, dst2, h_h, hs_o, hd_o,
          srcv, dstv, a0, b0, a1, b1, g0, g1, st0, st1):
        wid = lax.axis_index("s") * _NC + lax.axis_index("c")
        cb = wid * _CPW

        pltpu.sync_copy(src2.at[pl.ds(cb, _CPW)], srcv)
        pltpu.sync_copy(dst2.at[pl.ds(cb, _CPW)], dstv)

        def g_issue(jj, sa, sb, sem):
            pltpu.async_copy(h_h.at[srcv.at[jj]], sa, sem)
            pltpu.async_copy(h_h.at[dstv.at[jj]], sb, sem)

        def g_wait(jj, sa, sb, sem):
            pltpu.make_async_copy(h_h.at[srcv.at[jj]], sa, sem).wait()
            pltpu.make_async_copy(h_h.at[dstv.at[jj]], sb, sem).wait()

        def s_issue(jj, sa, sb, sem):
            base = (cb + jj) * _C
            pltpu.async_copy(sa, hs_o.at[pl.ds(base, _C)], sem)
            pltpu.async_copy(sb, hd_o.at[pl.ds(base, _C)], sem)

        def s_wait(jj, sa, sb, sem):
            base = (cb + jj) * _C
            pltpu.make_async_copy(sa, hs_o.at[pl.ds(base, _C)], sem).wait()
            pltpu.make_async_copy(sb, hd_o.at[pl.ds(base, _C)], sem).wait()

        g_issue(0, a0, b0, g0)

        def body(j, carry):
            j0 = 2 * j
            j1 = 2 * j + 1

            @pl.when(j > 0)
            def _():
                s_wait(j0 - 1, a1, b1, st1)

            g_issue(j1, a1, b1, g1)
            g_wait(j0, a0, b0, g0)
            s_issue(j0, a0, b0, st0)

            @pl.when(j < _HCPW - 1)
            def _():
                s_wait(j0, a0, b0, st0)
                g_issue(j0 + 2, a0, b0, g0)

            g_wait(j1, a1, b1, g1)
            s_issue(j1, a1, b1, st1)
            return carry

        lax.fori_loop(0, _HCPW, body, 0)
        s_wait(_CPW - 2, a0, b0, st0)
        s_wait(_CPW - 1, a1, b1, st1)

    return k


@functools.cache
def _scatter2():
    @functools.partial(
        pl.kernel,
        out_type=(_f32((_N, _H)), _f32((_N, _H))),
        mesh=_sc_mesh(),
        scratch_types=[
            pltpu.VMEM((_CPS, _C), jnp.int32),
            pltpu.VMEM((_C, _H), jnp.float32),
            pltpu.VMEM((_C, _H), jnp.float32),
            pltpu.VMEM_SHARED((_NP, _H), jnp.float32),
            pltpu.SemaphoreType.DMA,
            pltpu.SemaphoreType.DMA,
        ],
    )
    def k(msg_h, sig_h, dst2, zeros_h, num_o, den_o,
          idxv, v0, v1, accum, l0, l1):
        c = lax.axis_index("c")
        s = lax.axis_index("s")
        sb = s * _CPS

        pltpu.sync_copy(zeros_h, accum.at[pl.ds(s * _RPS, _RPS)])
        pltpu.sync_copy(dst2.at[pl.ds(sb, _CPS)], idxv)
        plsc.subcore_barrier()

        def run(val_h):
            def v_issue(jj, v, sem):
                base = (sb + jj) * _C
                pltpu.async_copy(val_h.at[pl.ds(base, _C)], v, sem)

            def v_wait(jj, v, sem):
                base = (sb + jj) * _C
                pltpu.make_async_copy(
                    val_h.at[pl.ds(base, _C)], v, sem).wait()

            v_issue(0, v0, l0)

            def body(j, carry):
                j0 = 2 * j
                j1 = 2 * j + 1
                v_issue(j1, v1, l1)
                v_wait(j0, v0, l0)
                pltpu.sync_copy(v0, accum.at[idxv.at[j0]], add=True)

                @pl.when(j < _HCPS - 1)
                def _():
                    v_issue(j0 + 2, v0, l0)

                v_wait(j1, v1, l1)
                pltpu.sync_copy(v1, accum.at[idxv.at[j1]], add=True)
                return carry

            lax.fori_loop(0, _HCPS, body, 0)

        @pl.when(c == 0)
        def _():
            run(msg_h)

        @pl.when(c == 1)
        def _():
            run(sig_h)

        plsc.subcore_barrier()

        last = _N - _RPS * (_NS - 1)

        @pl.when(jnp.logical_and(c == 0, s < _NS - 1))
        def _():
            pltpu.sync_copy(accum.at[pl.ds(s * _RPS, _RPS)],
                            num_o.at[pl.ds(s * _RPS, _RPS)])

        @pl.when(jnp.logical_and(c == 0, s == _NS - 1))
        def _():
            pltpu.sync_copy(accum.at[pl.ds((_NS - 1) * _RPS, last)],
                            num_o.at[pl.ds((_NS - 1) * _RPS, last)])

        @pl.when(jnp.logical_and(c == 1, s < _NS - 1))
        def _():
            pltpu.sync_copy(accum.at[pl.ds(s * _RPS, _RPS)],
                            den_o.at[pl.ds(s * _RPS, _RPS)])

        @pl.when(jnp.logical_and(c == 1, s == _NS - 1))
        def _():
            pltpu.sync_copy(accum.at[pl.ds((_NS - 1) * _RPS, last)],
                            den_o.at[pl.ds((_NS - 1) * _RPS, last)])

    return k


def _row_spec(b, w):
    return pl.BlockSpec((b, w), lambda i: (i, 0))


def _col_spec(b, w, col):
    return pl.BlockSpec((b, w), lambda i, _c=col: (i, _c))


def _full_spec(r, w):
    return pl.BlockSpec((r, w), lambda i: (0, 0))


def _h0_body(pe_r, w_r, b_r, out_r):
    out_r[...] = (jnp.dot(pe_r[...], w_r[...],
                          preferred_element_type=jnp.float32) + b_r[...])


@functools.cache
def _h0_call():
    return pl.pallas_call(
        _h0_body,
        grid=(_GN,),
        in_specs=[_row_spec(_BN, 10), _full_spec(10, _H), _full_spec(1, _H)],
        out_specs=_row_spec(_BN, _H),
        out_shape=_f32((_N, _H)),
    )


def _eh0_body(e_r, w1_r, b1_r, w2_r, b2_r, out_r):
    t = jnp.dot(e_r[...], w1_r[...], preferred_element_type=jnp.float32)
    t = jnp.maximum(t + b1_r[...], 0.0)
    out_r[...] = (jnp.dot(t, w2_r[...], preferred_element_type=jnp.float32)
                  + b2_r[...])


@functools.cache
def _eh0_call():
    return pl.pallas_call(
        _eh0_body,
        grid=(_GE,),
        in_specs=[_row_spec(_BE, 16), _full_spec(16, 16), _full_spec(1, 16),
                  _full_spec(16, _H), _full_spec(1, _H)],
        out_specs=_row_spec(_BE, _H),
        out_shape=_f32((_EP, _H)),
    )


def _node_mm_body(h_r, w_r, b_r, a_o, d_o, b_o, ew_o):
    r = jnp.dot(h_r[...], w_r[...], preferred_element_type=jnp.float32)
    r = r + b_r[...]
    a_o[...] = r[:, 0 * _H:1 * _H]
    d_o[...] = r[:, 1 * _H:2 * _H]
    b_o[...] = r[:, 2 * _H:3 * _H]
    ew_o[...] = r[:, 3 * _H:4 * _H]


@functools.cache
def _node_mm_call():
    return pl.pallas_call(
        _node_mm_body,
        grid=(_GN,),
        in_specs=[_row_spec(_BN, _H), _full_spec(_H, 4 * _H),
                  _full_spec(1, 4 * _H)],
        out_specs=tuple(_row_spec(_BN, _H) for _ in range(4)),
        out_shape=tuple(_f32((_N, _H)) for _ in range(4)),
    )


def _edge1_body(eh_r, cw_r, cb_r, dhs_r, ehd_r, enew_o, stats_o):
    i = pl.program_id(0)
    v = jnp.dot(eh_r[...], cw_r[...], preferred_element_type=jnp.float32)
    v = v + cb_r[...] + dhs_r[...] + ehd_r[...]
    enew_o[...] = v
    rows = lax.broadcasted_iota(jnp.int32, (_BE, _H), 0) + i * _BE
    vm = jnp.where(rows < _E, v, 0.0)
    st = jnp.concatenate(
        [jnp.sum(vm, axis=0, keepdims=True),
         jnp.sum(vm * vm, axis=0, keepdims=True),
         jnp.zeros((6, _H), jnp.float32)], axis=0)

    @pl.when(i == 0)
    def _():
        stats_o[...] = st

    @pl.when(i > 0)
    def _():
        stats_o[...] += st


@functools.cache
def _edge1_call():
    return pl.pallas_call(
        _edge1_body,
        grid=(_GE,),
        in_specs=[_row_spec(_BE, _H), _full_spec(_H, _H), _full_spec(1, _H),
                  _row_spec(_BE, _H), _row_spec(_BE, _H)],
        out_specs=(_row_spec(_BE, _H), _full_spec(8, _H)),
        out_shape=(_f32((_EP, _H)), _f32((8, _H))),
    )


def _edge2_body(eh_r, enew_r, bhs_r, st_r, g_r, b_r, ehn_o, sig_o, msg_o):
    st = st_r[...]
    m = st[0:1, :] * (1.0 / _E)
    var = st[1:2, :] * (1.0 / _E) - m * m
    inv = lax.rsqrt(var + 1e-5)
    xb = g_r[...] * (enew_r[...] - m) * inv + b_r[...]
    ehn = eh_r[...] + jnp.maximum(xb, 0.0)
    sig = jax.nn.sigmoid(ehn)
    ehn_o[...] = ehn
    sig_o[...] = sig
    msg_o[...] = sig * bhs_r[...]


@functools.cache
def _edge2_call():
    return pl.pallas_call(
        _edge2_body,
        grid=(_GE,),
        in_specs=[_row_spec(_BE, _H), _row_spec(_BE, _H),
                  _row_spec(_BE, _H),
                  _full_spec(8, _H), _full_spec(1, _H), _full_spec(1, _H)],
        out_specs=tuple(_row_spec(_BE, _H) for _ in range(3)),
        out_shape=tuple(_f32((_EP, _H)) for _ in range(3)),
    )


def _node1_body(ah_r, num_r, den_r, t_o, stats_o):
    i = pl.program_id(0)
    v = ah_r[...] + num_r[...] / (den_r[...] + 1e-6)
    t_o[...] = v
    st = jnp.concatenate(
        [jnp.sum(v, axis=0, keepdims=True),
         jnp.sum(v * v, axis=0, keepdims=True),
         jnp.zeros((6, _H), jnp.float32)], axis=0)

    @pl.when(i == 0)
    def _():
        stats_o[...] = st

    @pl.when(i > 0)
    def _():
        stats_o[...] += st


@functools.cache
def _node1_call():
    return pl.pallas_call(
        _node1_body,
        grid=(_GN,),
        in_specs=[_row_spec(_BN, _H)] * 3,
        out_specs=(_row_spec(_BN, _H), _full_spec(8, _H)),
        out_shape=(_f32((_N, _H)), _f32((8, _H))),
    )


def _node2_body(h_r, t_r, st_r, g_r, b_r, h_o):
    st = st_r[...]
    m = st[0:1, :] * (1.0 / _N)
    var = st[1:2, :] * (1.0 / _N) - m * m
    inv = lax.rsqrt(var + 1e-5)
    xb = g_r[...] * (t_r[...] - m) * inv + b_r[...]
    h_o[...] = h_r[...] + jnp.maximum(xb, 0.0)


@functools.cache
def _node2_call():
    return pl.pallas_call(
        _node2_body,
        grid=(_GN,),
        in_specs=[_row_spec(_BN, _H), _row_spec(_BN, _H), _full_spec(8, _H),
                  _full_spec(1, _H), _full_spec(1, _H)],
        out_specs=_row_spec(_BN, _H),
        out_shape=_f32((_N, _H)),
    )


def _final_body(hs_r, hd_r, eh_r, pa_r, pb_r, pc_r, p1b_r, p2_r, p2b_r,
                out_o):
    z = jnp.dot(hs_r[...], pa_r[...], preferred_element_type=jnp.float32)
    z = z + jnp.dot(hd_r[...], pb_r[...], preferred_element_type=jnp.float32)
    z = z + jnp.dot(eh_r[...], pc_r[...], preferred_element_type=jnp.float32)
    z = jnp.maximum(z + p1b_r[...], 0.0)
    out_o[...] = (jnp.dot(z, p2_r[...], preferred_element_type=jnp.float32)
                  + p2b_r[...])


@functools.cache
def _final_call():
    return pl.pallas_call(
        _final_body,
        grid=(_GE,),
        in_specs=[_row_spec(_BE, _H)] * 3
        + [_full_spec(_H, _H)] * 3
        + [_full_spec(1, _H), _full_spec(_H, 1), _full_spec(1, 1)],
        out_specs=_row_spec(_BE, 1),
        out_shape=_f32((_EP, 1)),
    )


def kernel(edge_index, x, e, pe, pe_W, pe_b, e1_W, e1_b, e2_W, e2_b,
           A_W, A_b, B_W, B_b, C_W, C_b, D_W, D_b, Ew_W, Ew_b,
           bn_h_g, bn_h_b, bn_e_g, bn_e_b, p1_W, p1_b, p2_W, p2_b):
    pad = _EP - _E
    src2 = jnp.concatenate(
        [edge_index[0], jnp.zeros((pad,), jnp.int32)]).reshape(_NCH, _C)
    dstg2 = jnp.concatenate(
        [edge_index[1], jnp.zeros((pad,), jnp.int32)]).reshape(_NCH, _C)
    dump = _DUMP + (jnp.arange(pad, dtype=jnp.int32) % (_NP - _N))
    dsts2 = jnp.concatenate([edge_index[1], dump]).reshape(_NCH, _C)
    e_p = jnp.concatenate([e, jnp.zeros((pad, 16), jnp.float32)], axis=0)

    h = _h0_call()(pe, pe_W, pe_b.reshape(1, _H))
    eh = _eh0_call()(e_p, e1_W, e1_b.reshape(1, 16), e2_W,
                     e2_b.reshape(1, _H))

    W4 = jnp.concatenate([A_W, D_W, B_W, Ew_W], axis=2)
    b4 = jnp.concatenate([A_b, D_b, B_b, Ew_b], axis=1)
    zeros_n = jnp.zeros((_RPS, _H), jnp.float32)

    for l in range(_L):
        Ah, Dh, Bh, Ewh = _node_mm_call()(h, W4[l], b4[l].reshape(1, 4 * _H))
        dhs, bhs, ehd = _gather3()(src2, dstg2, Dh, Bh, Ewh)
        enew, est = _edge1_call()(eh, C_W[l], C_b[l].reshape(1, _H), dhs, ehd)
        eh, sig, msg = _edge2_call()(eh, enew, bhs, est,
                                     bn_e_g[l].reshape(1, _H),
                                     bn_e_b[l].reshape(1, _H))
        num, den = _scatter2()(msg, sig, dsts2, zeros_n)
        t, nst = _node1_call()(Ah, num, den)
        h = _node2_call()(h, t, nst, bn_h_g[l].reshape(1, _H),
                          bn_h_b[l].reshape(1, _H))

    hs, hd = _gather2()(src2, dstg2, h)
    scores = _final_call()(hs, hd, eh,
                           p1_W[0 * _H:1 * _H], p1_W[1 * _H:2 * _H],
                           p1_W[2 * _H:3 * _H], p1_b.reshape(1, _H),
                           p2_W, p2_b.reshape(1, 1))
    return scores[:_E]


# speedup vs baseline: 1.2560x; 1.0947x over previous
import functools

import jax
import jax.numpy as jnp
from jax import lax
from jax.experimental import pallas as pl
from jax.experimental.pallas import tpu as pltpu
from jax.experimental.pallas import tpu_sc as plsc

_N = 10000
_E = 160000
_H = 128
_L = 4

_NC = 2
_NS = 16
_W = _NC * _NS

_C = 128
_EP = 163840
_NCH = _EP // _C
_CPW = _NCH // _W
_HCPW = _CPW // 2
_CPS = _NCH // _NS
_HCPS = _CPS // 2
_RPS = 640
_NP = _RPS * _NS
_DUMP = _N

_BE = 2048
_BN = 2000
_GE = _EP // _BE
_GN = _N // _BN


def _f32(shape):
    return jax.ShapeDtypeStruct(shape, jnp.float32)


def _sc_mesh():
    return plsc.VectorSubcoreMesh(
        core_axis_name="c", subcore_axis_name="s", num_cores=_NC,
        num_subcores=_NS)


@functools.cache
def _gather3():
    @functools.partial(
        pl.kernel,
        out_type=(_f32((_EP, _H)), _f32((_EP, _H)), _f32((_EP, _H))),
        mesh=_sc_mesh(),
        scratch_types=[
            pltpu.VMEM((_CPW, _C), jnp.int32),
            pltpu.VMEM((_CPW, _C), jnp.int32),
            pltpu.VMEM((_C, _H), jnp.float32),
            pltpu.VMEM((_C, _H), jnp.float32),
            pltpu.VMEM((_C, _H), jnp.float32),
            pltpu.VMEM((_C, _H), jnp.float32),
            pltpu.VMEM((_C, _H), jnp.float32),
            pltpu.VMEM((_C, _H), jnp.float32),
            pltpu.SemaphoreType.DMA,
            pltpu.SemaphoreType.DMA,
            pltpu.SemaphoreType.DMA,
            pltpu.SemaphoreType.DMA,
        ],
    )
    def k(src2, dst2, d_h, b_h, e_h, dhs_o, bhs_o, ehd_o,
          srcv, dstv, d0, b0, e0, d1, b1, e1, g0, g1, st0, st1):
        wid = lax.axis_index("s") * _NC + lax.axis_index("c")
        cb = wid * _CPW

        pltpu.sync_copy(src2.at[pl.ds(cb, _CPW)], srcv)
        pltpu.sync_copy(dst2.at[pl.ds(cb, _CPW)], dstv)

        def g_issue(jj, sd, sb, se, sem):
            pltpu.async_copy(d_h.at[srcv.at[jj]], sd, sem)
            pltpu.async_copy(b_h.at[srcv.at[jj]], sb, sem)
            pltpu.async_copy(e_h.at[dstv.at[jj]], se, sem)

        def g_wait(jj, sd, sb, se, sem):
            pltpu.make_async_copy(d_h.at[srcv.at[jj]], sd, sem).wait()
            pltpu.make_async_copy(b_h.at[srcv.at[jj]], sb, sem).wait()
            pltpu.make_async_copy(e_h.at[dstv.at[jj]], se, sem).wait()

        def s_issue(jj, sd, sb, se, sem):
            base = (cb + jj) * _C
            pltpu.async_copy(sd, dhs_o.at[pl.ds(base, _C)], sem)
            pltpu.async_copy(sb, bhs_o.at[pl.ds(base, _C)], sem)
            pltpu.async_copy(se, ehd_o.at[pl.ds(base, _C)], sem)

        def s_wait(jj, sd, sb, se, sem):
            base = (cb + jj) * _C
            pltpu.make_async_copy(sd, dhs_o.at[pl.ds(base, _C)], sem).wait()
            pltpu.make_async_copy(sb, bhs_o.at[pl.ds(base, _C)], sem).wait()
            pltpu.make_async_copy(se, ehd_o.at[pl.ds(base, _C)], sem).wait()

        g_issue(0, d0, b0, e0, g0)

        def body(j, carry):
            j0 = 2 * j
            j1 = 2 * j + 1

            @pl.when(j > 0)
            def _():
                s_wait(j0 - 1, d1, b1, e1, st1)

            g_issue(j1, d1, b1, e1, g1)
            g_wait(j0, d0, b0, e0, g0)
            s_issue(j0, d0, b0, e0, st0)

            @pl.when(j < _HCPW - 1)
            def _():
                s_wait(j0, d0, b0, e0, st0)
                g_issue(j0 + 2, d0, b0, e0, g0)

            g_wait(j1, d1, b1, e1, g1)
            s_issue(j1, d1, b1, e1, st1)
            return carry

        lax.fori_loop(0, _HCPW, body, 0)
        s_wait(_CPW - 2, d0, b0, e0, st0)
        s_wait(_CPW - 1, d1, b1, e1, st1)

    return k


@functools.cache
def _gather2():
    @functools.partial(
        pl.kernel,
        out_type=(_f32((_EP, _H)), _f32((_EP, _H))),
        mesh=_sc_mesh(),
        scratch_types=[
            pltpu.VMEM((_CPW, _C), jnp.int32),
            pltpu.VMEM((_CPW, _C), jnp.int32),
            pltpu.VMEM((_C, _H), jnp.float32),
            pltpu.VMEM((_C, _H), jnp.float32),
            pltpu.VMEM((_C, _H), jnp.float32),
            pltpu.VMEM((_C, _H), jnp.float32),
            pltpu.SemaphoreType.DMA,
            pltpu.SemaphoreType.DMA,
            pltpu.SemaphoreType.DMA,
            pltpu.SemaphoreType.DMA,
        ],
    )
    def k(src2, dst2, h_h, hs_o, hd_o,
          srcv, dstv, a0, b0, a1, b1, g0, g1, st0, st1):
        wid = lax.axis_index("s") * _NC + lax.axis_index("c")
        cb = wid * _CPW

        pltpu.sync_copy(src2.at[pl.ds(cb, _CPW)], srcv)
        pltpu.sync_copy(dst2.at[pl.ds(cb, _CPW)], dstv)

        def g_issue(jj, sa, sb, sem):
            pltpu.async_copy(h_h.at[srcv.at[jj]], sa, sem)
            pltpu.async_copy(h_h.at[dstv.at[jj]], sb, sem)

        def g_wait(jj, sa, sb, sem):
            pltpu.make_async_copy(h_h.at[srcv.at[jj]], sa, sem).wait()
            pltpu.make_async_copy(h_h.at[dstv.at[jj]], sb, sem).wait()

        def s_issue(jj, sa, sb, sem):
            base = (cb + jj) * _C
            pltpu.async_copy(sa, hs_o.at[pl.ds(base, _C)], sem)
            pltpu.async_copy(sb, hd_o.at[pl.ds(base, _C)], sem)

        def s_wait(jj, sa, sb, sem):
            base = (cb + jj) * _C
            pltpu.make_async_copy(sa, hs_o.at[pl.ds(base, _C)], sem).wait()
            pltpu.make_async_copy(sb, hd_o.at[pl.ds(base, _C)], sem).wait()

        g_issue(0, a0, b0, g0)

        def body(j, carry):
            j0 = 2 * j
            j1 = 2 * j + 1

            @pl.when(j > 0)
            def _():
                s_wait(j0 - 1, a1, b1, st1)

            g_issue(j1, a1, b1, g1)
            g_wait(j0, a0, b0, g0)
            s_issue(j0, a0, b0, st0)

            @pl.when(j < _HCPW - 1)
            def _():
                s_wait(j0, a0, b0, st0)
                g_issue(j0 + 2, a0, b0, g0)

            g_wait(j1, a1, b1, g1)
            s_issue(j1, a1, b1, st1)
            return carry

        lax.fori_loop(0, _HCPW, body, 0)
        s_wait(_CPW - 2, a0, b0, st0)
        s_wait(_CPW - 1, a1, b1, st1)

    return k


@functools.cache
def _scatter2():
    @functools.partial(
        pl.kernel,
        out_type=(_f32((_N, _H)), _f32((_N, _H))),
        mesh=_sc_mesh(),
        scratch_types=[
            pltpu.VMEM((_CPS, _C), jnp.int32),
            pltpu.VMEM((_C, _H), jnp.float32),
            pltpu.VMEM((_C, _H), jnp.float32),
            pltpu.VMEM_SHARED((_NP, _H), jnp.float32),
            pltpu.SemaphoreType.DMA,
            pltpu.SemaphoreType.DMA,
        ],
    )
    def k(msg_h, sig_h, dst2, zeros_h, num_o, den_o,
          idxv, v0, v1, accum, l0, l1):
        c = lax.axis_index("c")
        s = lax.axis_index("s")
        sb = s * _CPS

        pltpu.sync_copy(zeros_h, accum.at[pl.ds(s * _RPS, _RPS)])
        pltpu.sync_copy(dst2.at[pl.ds(sb, _CPS)], idxv)
        plsc.subcore_barrier()

        def run(val_h):
            def v_issue(jj, v, sem):
                base = (sb + jj) * _C
                pltpu.async_copy(val_h.at[pl.ds(base, _C)], v, sem)

            def v_wait(jj, v, sem):
                base = (sb + jj) * _C
                pltpu.make_async_copy(
                    val_h.at[pl.ds(base, _C)], v, sem).wait()

            v_issue(0, v0, l0)

            def body(j, carry):
                j0 = 2 * j
                j1 = 2 * j + 1
                v_issue(j1, v1, l1)
                v_wait(j0, v0, l0)
                pltpu.sync_copy(v0, accum.at[idxv.at[j0]], add=True)

                @pl.when(j < _HCPS - 1)
                def _():
                    v_issue(j0 + 2, v0, l0)

                v_wait(j1, v1, l1)
                pltpu.sync_copy(v1, accum.at[idxv.at[j1]], add=True)
                return carry

            lax.fori_loop(0, _HCPS, body, 0)

        @pl.when(c == 0)
        def _():
            run(msg_h)

        @pl.when(c == 1)
        def _():
            run(sig_h)

        plsc.subcore_barrier()

        last = _N - _RPS * (_NS - 1)

        @pl.when(jnp.logical_and(c == 0, s < _NS - 1))
        def _():
            pltpu.sync_copy(accum.at[pl.ds(s * _RPS, _RPS)],
                            num_o.at[pl.ds(s * _RPS, _RPS)])

        @pl.when(jnp.logical_and(c == 0, s == _NS - 1))
        def _():
            pltpu.sync_copy(accum.at[pl.ds((_NS - 1) * _RPS, last)],
                            num_o.at[pl.ds((_NS - 1) * _RPS, last)])

        @pl.when(jnp.logical_and(c == 1, s < _NS - 1))
        def _():
            pltpu.sync_copy(accum.at[pl.ds(s * _RPS, _RPS)],
                            den_o.at[pl.ds(s * _RPS, _RPS)])

        @pl.when(jnp.logical_and(c == 1, s == _NS - 1))
        def _():
            pltpu.sync_copy(accum.at[pl.ds((_NS - 1) * _RPS, last)],
                            den_o.at[pl.ds((_NS - 1) * _RPS, last)])

    return k


def _row_spec(b, w):
    return pl.BlockSpec((b, w), lambda i: (i, 0))


def _col_spec(b, w, col):
    return pl.BlockSpec((b, w), lambda i, _c=col: (i, _c))


def _full_spec(r, w):
    return pl.BlockSpec((r, w), lambda i: (0, 0))


def _h0_body(pe_r, w_r, b_r, out_r):
    out_r[...] = (jnp.dot(pe_r[...], w_r[...],
                          preferred_element_type=jnp.float32) + b_r[...])


@functools.cache
def _h0_call():
    return pl.pallas_call(
        _h0_body,
        grid=(_GN,),
        in_specs=[_row_spec(_BN, 10), _full_spec(10, _H), _full_spec(1, _H)],
        out_specs=_row_spec(_BN, _H),
        out_shape=_f32((_N, _H)),
    )


def _eh0_body(e_r, w1_r, b1_r, w2_r, b2_r, out_r):
    t = jnp.dot(e_r[...], w1_r[...], preferred_element_type=jnp.float32)
    t = jnp.maximum(t + b1_r[...], 0.0)
    out_r[...] = (jnp.dot(t, w2_r[...], preferred_element_type=jnp.float32)
                  + b2_r[...])


@functools.cache
def _eh0_call():
    return pl.pallas_call(
        _eh0_body,
        grid=(_GE,),
        in_specs=[_row_spec(_BE, 16), _full_spec(16, 16), _full_spec(1, 16),
                  _full_spec(16, _H), _full_spec(1, _H)],
        out_specs=_row_spec(_BE, _H),
        out_shape=_f32((_EP, _H)),
    )


def _node_mm_body(h_r, w_r, b_r, a_o, d_o, b_o, ew_o):
    r = jnp.dot(h_r[...], w_r[...], preferred_element_type=jnp.float32)
    r = r + b_r[...]
    a_o[...] = r[:, 0 * _H:1 * _H]
    d_o[...] = r[:, 1 * _H:2 * _H]
    b_o[...] = r[:, 2 * _H:3 * _H]
    ew_o[...] = r[:, 3 * _H:4 * _H]


@functools.cache
def _node_mm_call():
    return pl.pallas_call(
        _node_mm_body,
        grid=(_GN,),
        in_specs=[_row_spec(_BN, _H), _full_spec(_H, 4 * _H),
                  _full_spec(1, 4 * _H)],
        out_specs=tuple(_row_spec(_BN, _H) for _ in range(4)),
        out_shape=tuple(_f32((_N, _H)) for _ in range(4)),
    )


def _edge1_body(eh_r, cw_r, cb_r, dhs_r, ehd_r, enew_o, stats_o):
    i = pl.program_id(0)
    v = jnp.dot(eh_r[...], cw_r[...], preferred_element_type=jnp.float32)
    v = v + cb_r[...] + dhs_r[...] + ehd_r[...]
    enew_o[...] = v
    rows = lax.broadcasted_iota(jnp.int32, (_BE, _H), 0) + i * _BE
    vm = jnp.where(rows < _E, v, 0.0)
    st = jnp.concatenate(
        [jnp.sum(vm, axis=0, keepdims=True),
         jnp.sum(vm * vm, axis=0, keepdims=True),
         jnp.zeros((6, _H), jnp.float32)], axis=0)

    @pl.when(i == 0)
    def _():
        stats_o[...] = st

    @pl.when(i > 0)
    def _():
        stats_o[...] += st


@functools.cache
def _edge1_call():
    return pl.pallas_call(
        _edge1_body,
        grid=(_GE,),
        in_specs=[_row_spec(_BE, _H), _full_spec(_H, _H), _full_spec(1, _H),
                  _row_spec(_BE, _H), _row_spec(_BE, _H)],
        out_specs=(_row_spec(_BE, _H), _full_spec(8, _H)),
        out_shape=(_f32((_EP, _H)), _f32((8, _H))),
    )


def _edge2_body(eh_r, enew_r, bhs_r, st_r, g_r, b_r, ehn_o, sig_o, msg_o):
    st = st_r[...]
    m = st[0:1, :] * (1.0 / _E)
    var = st[1:2, :] * (1.0 / _E) - m * m
    inv = lax.rsqrt(var + 1e-5)
    xb = g_r[...] * (enew_r[...] - m) * inv + b_r[...]
    ehn = eh_r[...] + jnp.maximum(xb, 0.0)
    sig = jax.nn.sigmoid(ehn)
    ehn_o[...] = ehn
    sig_o[...] = sig
    msg_o[...] = sig * bhs_r[...]


@functools.cache
def _edge2_call():
    return pl.pallas_call(
        _edge2_body,
        grid=(_GE,),
        in_specs=[_row_spec(_BE, _H), _row_spec(_BE, _H),
                  _row_spec(_BE, _H),
                  _full_spec(8, _H), _full_spec(1, _H), _full_spec(1, _H)],
        out_specs=tuple(_row_spec(_BE, _H) for _ in range(3)),
        out_shape=tuple(_f32((_EP, _H)) for _ in range(3)),
    )


def _node1_body(ah_r, num_r, den_r, t_o, stats_o):
    i = pl.program_id(0)
    v = ah_r[...] + num_r[...] / (den_r[...] + 1e-6)
    t_o[...] = v
    st = jnp.concatenate(
        [jnp.sum(v, axis=0, keepdims=True),
         jnp.sum(v * v, axis=0, keepdims=True),
         jnp.zeros((6, _H), jnp.float32)], axis=0)

    @pl.when(i == 0)
    def _():
        stats_o[...] = st

    @pl.when(i > 0)
    def _():
        stats_o[...] += st


@functools.cache
def _node1_call():
    return pl.pallas_call(
        _node1_body,
        grid=(_GN,),
        in_specs=[_row_spec(_BN, _H)] * 3,
        out_specs=(_row_spec(_BN, _H), _full_spec(8, _H)),
        out_shape=(_f32((_N, _H)), _f32((8, _H))),
    )


def _node2_body(h_r, t_r, st_r, g_r, b_r, h_o):
    st = st_r[...]
    m = st[0:1, :] * (1.0 / _N)
    var = st[1:2, :] * (1.0 / _N) - m * m
    inv = lax.rsqrt(var + 1e-5)
    xb = g_r[...] * (t_r[...] - m) * inv + b_r[...]
    h_o[...] = h_r[...] + jnp.maximum(xb, 0.0)


@functools.cache
def _node2_call():
    return pl.pallas_call(
        _node2_body,
        grid=(_GN,),
        in_specs=[_row_spec(_BN, _H), _row_spec(_BN, _H), _full_spec(8, _H),
                  _full_spec(1, _H), _full_spec(1, _H)],
        out_specs=_row_spec(_BN, _H),
        out_shape=_f32((_N, _H)),
    )


def _final_body(hs_r, hd_r, eh_r, pa_r, pb_r, pc_r, p1b_r, p2_r, p2b_r,
                out_o):
    z = jnp.dot(hs_r[...], pa_r[...], preferred_element_type=jnp.float32)
    z = z + jnp.dot(hd_r[...], pb_r[...], preferred_element_type=jnp.float32)
    z = z + jnp.dot(eh_r[...], pc_r[...], preferred_element_type=jnp.float32)
    z = jnp.maximum(z + p1b_r[...], 0.0)
    out_o[...] = (jnp.dot(z, p2_r[...], preferred_element_type=jnp.float32)
                  + p2b_r[...])


@functools.cache
def _final_call():
    return pl.pallas_call(
        _final_body,
        grid=(_GE,),
        in_specs=[_row_spec(_BE, _H)] * 3
        + [_full_spec(_H, _H)] * 3
        + [_full_spec(1, _H), _full_spec(_H, 1), _full_spec(1, 1)],
        out_specs=_row_spec(_BE, 1),
        out_shape=_f32((_EP, 1)),
    )


def kernel(edge_index, x, e, pe, pe_W, pe_b, e1_W, e1_b, e2_W, e2_b,
           A_W, A_b, B_W, B_b, C_W, C_b, D_W, D_b, Ew_W, Ew_b,
           bn_h_g, bn_h_b, bn_e_g, bn_e_b, p1_W, p1_b, p2_W, p2_b):
    pad = _EP - _E
    src2 = jnp.concatenate(
        [edge_index[0], jnp.zeros((pad,), jnp.int32)]).reshape(_NCH, _C)
    dstg2 = jnp.concatenate(
        [edge_index[1], jnp.zeros((pad,), jnp.int32)]).reshape(_NCH, _C)
    dump = _DUMP + (jnp.arange(pad, dtype=jnp.int32) % (_NP - _N))
    dsts2 = jnp.concatenate([edge_index[1], dump]).reshape(_NCH, _C)
    e_p = jnp.concatenate([e, jnp.zeros((pad, 16), jnp.float32)], axis=0)

    h = _h0_call()(pe, pe_W, pe_b.reshape(1, _H))
    eh = _eh0_call()(e_p, e1_W, e1_b.reshape(1, 16), e2_W,
                     e2_b.reshape(1, _H))

    W4 = jnp.concatenate([A_W, D_W, B_W, Ew_W], axis=2)
    b4 = jnp.concatenate([A_b, D_b, B_b, Ew_b], axis=1)
    zeros_n = jnp.zeros((_RPS, _H), jnp.float32)

    for l in range(_L):
        Ah, Dh, Bh, Ewh = _node_mm_call()(h, W4[l], b4[l].reshape(1, 4 * _H))
        dhs, bhs, ehd = _gather3()(src2, dstg2, Dh, Bh, Ewh)
        enew, est = _edge1_call()(eh, C_W[l], C_b[l].reshape(1, _H), dhs, ehd)
        eh, sig, msg = _edge2_call()(eh, enew, bhs, est,
                                     bn_e_g[l].reshape(1, _H),
                                     bn_e_b[l].reshape(1, _H))
        num, den = _scatter2()(msg, sig, dsts2, zeros_n)
        t, nst = _node1_call()(Ah, num, den)
        h = _node2_call()(h, t, nst, bn_h_g[l].reshape(1, _H),
                          bn_h_b[l].reshape(1, _H))

    hs, hd = _gather2()(src2, dstg2, h)
    scores = _final_call()(hs, hd, eh,
                           p1_W[0 * _H:1 * _H], p1_W[1 * _H:2 * _H],
                           p1_W[2 * _H:3 * _H], p1_b.reshape(1, _H),
                           p2_W, p2_b.reshape(1, 1))
    return scores[:_E]


# speedup vs baseline: 1.2919x; 1.0286x over previous
import functools

import jax
import jax.numpy as jnp
from jax import lax
from jax.experimental import pallas as pl
from jax.experimental.pallas import tpu as pltpu
from jax.experimental.pallas import tpu_sc as plsc

_N = 10000
_E = 160000
_H = 128
_L = 4

_NC = 2
_NS = 16
_W = _NC * _NS

_C = 128
_EP = 163840
_NCH = _EP // _C
_CPW = _NCH // _W
_HCPW = _CPW // 2
_CPS = _NCH // _NS
_HCPS = _CPS // 2
_RPS = 640
_NP = _RPS * _NS
_DUMP = _N

_BE = 4096
_BN = 5000
_GE = _EP // _BE
_GN = _N // _BN


def _f32(shape):
    return jax.ShapeDtypeStruct(shape, jnp.float32)


def _sc_mesh():
    return plsc.VectorSubcoreMesh(
        core_axis_name="c", subcore_axis_name="s", num_cores=_NC,
        num_subcores=_NS)


@functools.cache
def _gather3():
    @functools.partial(
        pl.kernel,
        out_type=(_f32((_EP, _H)), _f32((_EP, _H)), _f32((_EP, _H))),
        mesh=_sc_mesh(),
        scratch_types=[
            pltpu.VMEM((_CPW, _C), jnp.int32),
            pltpu.VMEM((_CPW, _C), jnp.int32),
            pltpu.VMEM((_C, _H), jnp.float32),
            pltpu.VMEM((_C, _H), jnp.float32),
            pltpu.VMEM((_C, _H), jnp.float32),
            pltpu.VMEM((_C, _H), jnp.float32),
            pltpu.VMEM((_C, _H), jnp.float32),
            pltpu.VMEM((_C, _H), jnp.float32),
            pltpu.SemaphoreType.DMA,
            pltpu.SemaphoreType.DMA,
            pltpu.SemaphoreType.DMA,
            pltpu.SemaphoreType.DMA,
        ],
    )
    def k(src2, dst2, d_h, b_h, e_h, dhs_o, bhs_o, ehd_o,
          srcv, dstv, d0, b0, e0, d1, b1, e1, g0, g1, st0, st1):
        wid = lax.axis_index("s") * _NC + lax.axis_index("c")
        cb = wid * _CPW

        pltpu.sync_copy(src2.at[pl.ds(cb, _CPW)], srcv)
        pltpu.sync_copy(dst2.at[pl.ds(cb, _CPW)], dstv)

        def g_issue(jj, sd, sb, se, sem):
            pltpu.async_copy(d_h.at[srcv.at[jj]], sd, sem)
            pltpu.async_copy(b_h.at[srcv.at[jj]], sb, sem)
            pltpu.async_copy(e_h.at[dstv.at[jj]], se, sem)

        def g_wait(jj, sd, sb, se, sem):
            pltpu.make_async_copy(d_h.at[srcv.at[jj]], sd, sem).wait()
            pltpu.make_async_copy(b_h.at[srcv.at[jj]], sb, sem).wait()
            pltpu.make_async_copy(e_h.at[dstv.at[jj]], se, sem).wait()

        def s_issue(jj, sd, sb, se, sem):
            base = (cb + jj) * _C
            pltpu.async_copy(sd, dhs_o.at[pl.ds(base, _C)], sem)
            pltpu.async_copy(sb, bhs_o.at[pl.ds(base, _C)], sem)
            pltpu.async_copy(se, ehd_o.at[pl.ds(base, _C)], sem)

        def s_wait(jj, sd, sb, se, sem):
            base = (cb + jj) * _C
            pltpu.make_async_copy(sd, dhs_o.at[pl.ds(base, _C)], sem).wait()
            pltpu.make_async_copy(sb, bhs_o.at[pl.ds(base, _C)], sem).wait()
            pltpu.make_async_copy(se, ehd_o.at[pl.ds(base, _C)], sem).wait()

        g_issue(0, d0, b0, e0, g0)

        def body(j, carry):
            j0 = 2 * j
            j1 = 2 * j + 1

            @pl.when(j > 0)
            def _():
                s_wait(j0 - 1, d1, b1, e1, st1)

            g_issue(j1, d1, b1, e1, g1)
            g_wait(j0, d0, b0, e0, g0)
            s_issue(j0, d0, b0, e0, st0)

            @pl.when(j < _HCPW - 1)
            def _():
                s_wait(j0, d0, b0, e0, st0)
                g_issue(j0 + 2, d0, b0, e0, g0)

            g_wait(j1, d1, b1, e1, g1)
            s_issue(j1, d1, b1, e1, st1)
            return carry

        lax.fori_loop(0, _HCPW, body, 0)
        s_wait(_CPW - 2, d0, b0, e0, st0)
        s_wait(_CPW - 1, d1, b1, e1, st1)

    return k


@functools.cache
def _gather2():
    @functools.partial(
        pl.kernel,
        out_type=(_f32((_EP, _H)), _f32((_EP, _H))),
        mesh=_sc_mesh(),
        scratch_types=[
            pltpu.VMEM((_CPW, _C), jnp.int32),
            pltpu.VMEM((_CPW, _C), jnp.int32),
            pltpu.VMEM((_C, _H), jnp.float32),
            pltpu.VMEM((_C, _H), jnp.float32),
            pltpu.VMEM((_C, _H), jnp.float32),
            pltpu.VMEM((_C, _H), jnp.float32),
            pltpu.SemaphoreType.DMA,
            pltpu.SemaphoreType.DMA,
            pltpu.SemaphoreType.DMA,
            pltpu.SemaphoreType.DMA,
        ],
    )
    def k(src2, dst2, h_h, hs_o, hd_o,
          srcv, dstv, a0, b0, a1, b1, g0, g1, st0, st1):
        wid = lax.axis_index("s") * _NC + lax.axis_index("c")
        cb = wid * _CPW

        pltpu.sync_copy(src2.at[pl.ds(cb, _CPW)], srcv)
        pltpu.sync_copy(dst2.at[pl.ds(cb, _CPW)], dstv)

        def g_issue(jj, sa, sb, sem):
            pltpu.async_copy(h_h.at[srcv.at[jj]], sa, sem)
            pltpu.async_copy(h_h.at[dstv.at[jj]], sb, sem)

        def g_wait(jj, sa, sb, sem):
            pltpu.make_async_copy(h_h.at[srcv.at[jj]], sa, sem).wait()
            pltpu.make_async_copy(h_h.at[dstv.at[jj]], sb, sem).wait()

        def s_issue(jj, sa, sb, sem):
            base = (cb + jj) * _C
            pltpu.async_copy(sa, hs_o.at[pl.ds(base, _C)], sem)
            pltpu.async_copy(sb, hd_o.at[pl.ds(base, _C)], sem)

        def s_wait(jj, sa, sb, sem):
            base = (cb + jj) * _C
            pltpu.make_async_copy(sa, hs_o.at[pl.ds(base, _C)], sem).wait()
            pltpu.make_async_copy(sb, hd_o.at[pl.ds(base, _C)], sem).wait()

        g_issue(0, a0, b0, g0)

        def body(j, carry):
            j0 = 2 * j
            j1 = 2 * j + 1

            @pl.when(j > 0)
            def _():
                s_wait(j0 - 1, a1, b1, st1)

            g_issue(j1, a1, b1, g1)
            g_wait(j0, a0, b0, g0)
            s_issue(j0, a0, b0, st0)

            @pl.when(j < _HCPW - 1)
            def _():
                s_wait(j0, a0, b0, st0)
                g_issue(j0 + 2, a0, b0, g0)

            g_wait(j1, a1, b1, g1)
            s_issue(j1, a1, b1, st1)
            return carry

        lax.fori_loop(0, _HCPW, body, 0)
        s_wait(_CPW - 2, a0, b0, st0)
        s_wait(_CPW - 1, a1, b1, st1)

    return k


@functools.cache
def _scatter2():
    @functools.partial(
        pl.kernel,
        out_type=(_f32((_N, _H)), _f32((_N, _H))),
        mesh=_sc_mesh(),
        scratch_types=[
            pltpu.VMEM((_CPS, _C), jnp.int32),
            pltpu.VMEM((_C, _H), jnp.float32),
            pltpu.VMEM((_C, _H), jnp.float32),
            pltpu.VMEM_SHARED((_NP, _H), jnp.float32),
            pltpu.SemaphoreType.DMA,
            pltpu.SemaphoreType.DMA,
        ],
    )
    def k(msg_h, sig_h, dst2, zeros_h, num_o, den_o,
          idxv, v0, v1, accum, l0, l1):
        c = lax.axis_index("c")
        s = lax.axis_index("s")
        sb = s * _CPS

        pltpu.sync_copy(zeros_h, accum.at[pl.ds(s * _RPS, _RPS)])
        pltpu.sync_copy(dst2.at[pl.ds(sb, _CPS)], idxv)
        plsc.subcore_barrier()

        def run(val_h):
            def v_issue(jj, v, sem):
                base = (sb + jj) * _C
                pltpu.async_copy(val_h.at[pl.ds(base, _C)], v, sem)

            def v_wait(jj, v, sem):
                base = (sb + jj) * _C
                pltpu.make_async_copy(
                    val_h.at[pl.ds(base, _C)], v, sem).wait()

            v_issue(0, v0, l0)

            def body(j, carry):
                j0 = 2 * j
                j1 = 2 * j + 1
                v_issue(j1, v1, l1)
                v_wait(j0, v0, l0)
                pltpu.sync_copy(v0, accum.at[idxv.at[j0]], add=True)

                @pl.when(j < _HCPS - 1)
                def _():
                    v_issue(j0 + 2, v0, l0)

                v_wait(j1, v1, l1)
                pltpu.sync_copy(v1, accum.at[idxv.at[j1]], add=True)
                return carry

            lax.fori_loop(0, _HCPS, body, 0)

        @pl.when(c == 0)
        def _():
            run(msg_h)

        @pl.when(c == 1)
        def _():
            run(sig_h)

        plsc.subcore_barrier()

        last = _N - _RPS * (_NS - 1)

        @pl.when(jnp.logical_and(c == 0, s < _NS - 1))
        def _():
            pltpu.sync_copy(accum.at[pl.ds(s * _RPS, _RPS)],
                            num_o.at[pl.ds(s * _RPS, _RPS)])

        @pl.when(jnp.logical_and(c == 0, s == _NS - 1))
        def _():
            pltpu.sync_copy(accum.at[pl.ds((_NS - 1) * _RPS, last)],
                            num_o.at[pl.ds((_NS - 1) * _RPS, last)])

        @pl.when(jnp.logical_and(c == 1, s < _NS - 1))
        def _():
            pltpu.sync_copy(accum.at[pl.ds(s * _RPS, _RPS)],
                            den_o.at[pl.ds(s * _RPS, _RPS)])

        @pl.when(jnp.logical_and(c == 1, s == _NS - 1))
        def _():
            pltpu.sync_copy(accum.at[pl.ds((_NS - 1) * _RPS, last)],
                            den_o.at[pl.ds((_NS - 1) * _RPS, last)])

    return k


def _row_spec(b, w):
    return pl.BlockSpec((b, w), lambda i: (i, 0))


def _col_spec(b, w, col):
    return pl.BlockSpec((b, w), lambda i, _c=col: (i, _c))


def _full_spec(r, w):
    return pl.BlockSpec((r, w), lambda i: (0, 0))


def _h0_body(pe_r, w_r, b_r, out_r):
    out_r[...] = (jnp.dot(pe_r[...], w_r[...],
                          preferred_element_type=jnp.float32) + b_r[...])


@functools.cache
def _h0_call():
    return pl.pallas_call(
        _h0_body,
        grid=(_GN,),
        in_specs=[_row_spec(_BN, 10), _full_spec(10, _H), _full_spec(1, _H)],
        out_specs=_row_spec(_BN, _H),
        out_shape=_f32((_N, _H)),
    )


def _eh0_body(e_r, w1_r, b1_r, w2_r, b2_r, out_r):
    t = jnp.dot(e_r[...], w1_r[...], preferred_element_type=jnp.float32)
    t = jnp.maximum(t + b1_r[...], 0.0)
    out_r[...] = (jnp.dot(t, w2_r[...], preferred_element_type=jnp.float32)
                  + b2_r[...])


@functools.cache
def _eh0_call():
    return pl.pallas_call(
        _eh0_body,
        grid=(_GE,),
        in_specs=[_row_spec(_BE, 16), _full_spec(16, 16), _full_spec(1, 16),
                  _full_spec(16, _H), _full_spec(1, _H)],
        out_specs=_row_spec(_BE, _H),
        out_shape=_f32((_EP, _H)),
    )


def _node_mm_body(h_r, w_r, b_r, a_o, d_o, b_o, ew_o):
    r = jnp.dot(h_r[...], w_r[...], preferred_element_type=jnp.float32)
    r = r + b_r[...]
    a_o[...] = r[:, 0 * _H:1 * _H]
    d_o[...] = r[:, 1 * _H:2 * _H]
    b_o[...] = r[:, 2 * _H:3 * _H]
    ew_o[...] = r[:, 3 * _H:4 * _H]


@functools.cache
def _node_mm_call():
    return pl.pallas_call(
        _node_mm_body,
        grid=(_GN,),
        in_specs=[_row_spec(_BN, _H), _full_spec(_H, 4 * _H),
                  _full_spec(1, 4 * _H)],
        out_specs=tuple(_row_spec(_BN, _H) for _ in range(4)),
        out_shape=tuple(_f32((_N, _H)) for _ in range(4)),
    )


def _edge1_body(eh_r, cw_r, cb_r, dhs_r, ehd_r, enew_o, stats_o):
    i = pl.program_id(0)
    v = jnp.dot(eh_r[...], cw_r[...], preferred_element_type=jnp.float32)
    v = v + cb_r[...] + dhs_r[...] + ehd_r[...]
    enew_o[...] = v
    rows = lax.broadcasted_iota(jnp.int32, (_BE, _H), 0) + i * _BE
    vm = jnp.where(rows < _E, v, 0.0)
    st = jnp.concatenate(
        [jnp.sum(vm, axis=0, keepdims=True),
         jnp.sum(vm * vm, axis=0, keepdims=True),
         jnp.zeros((6, _H), jnp.float32)], axis=0)

    @pl.when(i == 0)
    def _():
        stats_o[...] = st

    @pl.when(i > 0)
    def _():
        stats_o[...] += st


@functools.cache
def _edge1_call():
    return pl.pallas_call(
        _edge1_body,
        grid=(_GE,),
        in_specs=[_row_spec(_BE, _H), _full_spec(_H, _H), _full_spec(1, _H),
                  _row_spec(_BE, _H), _row_spec(_BE, _H)],
        out_specs=(_row_spec(_BE, _H), _full_spec(8, _H)),
        out_shape=(_f32((_EP, _H)), _f32((8, _H))),
    )


def _edge2_body(eh_r, enew_r, bhs_r, st_r, g_r, b_r, ehn_o, sig_o, msg_o):
    st = st_r[...]
    m = st[0:1, :] * (1.0 / _E)
    var = st[1:2, :] * (1.0 / _E) - m * m
    inv = lax.rsqrt(var + 1e-5)
    xb = g_r[...] * (enew_r[...] - m) * inv + b_r[...]
    ehn = eh_r[...] + jnp.maximum(xb, 0.0)
    sig = jax.nn.sigmoid(ehn)
    ehn_o[...] = ehn
    sig_o[...] = sig
    msg_o[...] = sig * bhs_r[...]


@functools.cache
def _edge2_call():
    return pl.pallas_call(
        _edge2_body,
        grid=(_GE,),
        in_specs=[_row_spec(_BE, _H), _row_spec(_BE, _H),
                  _row_spec(_BE, _H),
                  _full_spec(8, _H), _full_spec(1, _H), _full_spec(1, _H)],
        out_specs=tuple(_row_spec(_BE, _H) for _ in range(3)),
        out_shape=tuple(_f32((_EP, _H)) for _ in range(3)),
    )


def _node1_body(ah_r, num_r, den_r, t_o, stats_o):
    i = pl.program_id(0)
    v = ah_r[...] + num_r[...] / (den_r[...] + 1e-6)
    t_o[...] = v
    st = jnp.concatenate(
        [jnp.sum(v, axis=0, keepdims=True),
         jnp.sum(v * v, axis=0, keepdims=True),
         jnp.zeros((6, _H), jnp.float32)], axis=0)

    @pl.when(i == 0)
    def _():
        stats_o[...] = st

    @pl.when(i > 0)
    def _():
        stats_o[...] += st


@functools.cache
def _node1_call():
    return pl.pallas_call(
        _node1_body,
        grid=(_GN,),
        in_specs=[_row_spec(_BN, _H)] * 3,
        out_specs=(_row_spec(_BN, _H), _full_spec(8, _H)),
        out_shape=(_f32((_N, _H)), _f32((8, _H))),
    )


def _node2_body(h_r, t_r, st_r, g_r, b_r, h_o):
    st = st_r[...]
    m = st[0:1, :] * (1.0 / _N)
    var = st[1:2, :] * (1.0 / _N) - m * m
    inv = lax.rsqrt(var + 1e-5)
    xb = g_r[...] * (t_r[...] - m) * inv + b_r[...]
    h_o[...] = h_r[...] + jnp.maximum(xb, 0.0)


@functools.cache
def _node2_call():
    return pl.pallas_call(
        _node2_body,
        grid=(_GN,),
        in_specs=[_row_spec(_BN, _H), _row_spec(_BN, _H), _full_spec(8, _H),
                  _full_spec(1, _H), _full_spec(1, _H)],
        out_specs=_row_spec(_BN, _H),
        out_shape=_f32((_N, _H)),
    )


def _final_body(hs_r, hd_r, eh_r, pa_r, pb_r, pc_r, p1b_r, p2_r, p2b_r,
                out_o):
    z = jnp.dot(hs_r[...], pa_r[...], preferred_element_type=jnp.float32)
    z = z + jnp.dot(hd_r[...], pb_r[...], preferred_element_type=jnp.float32)
    z = z + jnp.dot(eh_r[...], pc_r[...], preferred_element_type=jnp.float32)
    z = jnp.maximum(z + p1b_r[...], 0.0)
    out_o[...] = (jnp.dot(z, p2_r[...], preferred_element_type=jnp.float32)
                  + p2b_r[...])


@functools.cache
def _final_call():
    return pl.pallas_call(
        _final_body,
        grid=(_GE,),
        in_specs=[_row_spec(_BE, _H)] * 3
        + [_full_spec(_H, _H)] * 3
        + [_full_spec(1, _H), _full_spec(_H, 1), _full_spec(1, 1)],
        out_specs=_row_spec(_BE, 1),
        out_shape=_f32((_EP, 1)),
    )


def kernel(edge_index, x, e, pe, pe_W, pe_b, e1_W, e1_b, e2_W, e2_b,
           A_W, A_b, B_W, B_b, C_W, C_b, D_W, D_b, Ew_W, Ew_b,
           bn_h_g, bn_h_b, bn_e_g, bn_e_b, p1_W, p1_b, p2_W, p2_b):
    pad = _EP - _E
    src2 = jnp.concatenate(
        [edge_index[0], jnp.zeros((pad,), jnp.int32)]).reshape(_NCH, _C)
    dstg2 = jnp.concatenate(
        [edge_index[1], jnp.zeros((pad,), jnp.int32)]).reshape(_NCH, _C)
    dump = _DUMP + (jnp.arange(pad, dtype=jnp.int32) % (_NP - _N))
    dsts2 = jnp.concatenate([edge_index[1], dump]).reshape(_NCH, _C)
    e_p = jnp.concatenate([e, jnp.zeros((pad, 16), jnp.float32)], axis=0)

    h = _h0_call()(pe, pe_W, pe_b.reshape(1, _H))
    eh = _eh0_call()(e_p, e1_W, e1_b.reshape(1, 16), e2_W,
                     e2_b.reshape(1, _H))

    W4 = jnp.concatenate([A_W, D_W, B_W, Ew_W], axis=2)
    b4 = jnp.concatenate([A_b, D_b, B_b, Ew_b], axis=1)
    zeros_n = jnp.zeros((_RPS, _H), jnp.float32)

    for l in range(_L):
        Ah, Dh, Bh, Ewh = _node_mm_call()(h, W4[l], b4[l].reshape(1, 4 * _H))
        dhs, bhs, ehd = _gather3()(src2, dstg2, Dh, Bh, Ewh)
        enew, est = _edge1_call()(eh, C_W[l], C_b[l].reshape(1, _H), dhs, ehd)
        eh, sig, msg = _edge2_call()(eh, enew, bhs, est,
                                     bn_e_g[l].reshape(1, _H),
                                     bn_e_b[l].reshape(1, _H))
        num, den = _scatter2()(msg, sig, dsts2, zeros_n)
        t, nst = _node1_call()(Ah, num, den)
        h = _node2_call()(h, t, nst, bn_h_g[l].reshape(1, _H),
                          bn_h_b[l].reshape(1, _H))

    hs, hd = _gather2()(src2, dstg2, h)
    scores = _final_call()(hs, hd, eh,
                           p1_W[0 * _H:1 * _H], p1_W[1 * _H:2 * _H],
                           p1_W[2 * _H:3 * _H], p1_b.reshape(1, _H),
                           p2_W, p2_b.reshape(1, 1))
    return scores[:_E]


# speedup vs baseline: 1.2943x; 1.0019x over previous
import functools

import jax
import jax.numpy as jnp
from jax import lax
from jax.experimental import pallas as pl
from jax.experimental.pallas import tpu as pltpu
from jax.experimental.pallas import tpu_sc as plsc

_N = 10000
_E = 160000
_H = 128
_L = 4

_NC = 2
_NS = 16
_W = _NC * _NS

_C = 128
_EP = 163840
_NCH = _EP // _C
_CPW = _NCH // _W
_HCPW = _CPW // 2
_CPS = _NCH // _NS
_HCPS = _CPS // 2
_RPS = 640
_NP = _RPS * _NS
_DUMP = _N

_BE = 8192
_BN = 5000
_GE = _EP // _BE
_GN = _N // _BN


def _f32(shape):
    return jax.ShapeDtypeStruct(shape, jnp.float32)


def _sc_mesh():
    return plsc.VectorSubcoreMesh(
        core_axis_name="c", subcore_axis_name="s", num_cores=_NC,
        num_subcores=_NS)


@functools.cache
def _gather3():
    @functools.partial(
        pl.kernel,
        out_type=(_f32((_EP, _H)), _f32((_EP, _H)), _f32((_EP, _H))),
        mesh=_sc_mesh(),
        scratch_types=[
            pltpu.VMEM((_CPW, _C), jnp.int32),
            pltpu.VMEM((_CPW, _C), jnp.int32),
            pltpu.VMEM((_C, _H), jnp.float32),
            pltpu.VMEM((_C, _H), jnp.float32),
            pltpu.VMEM((_C, _H), jnp.float32),
            pltpu.VMEM((_C, _H), jnp.float32),
            pltpu.VMEM((_C, _H), jnp.float32),
            pltpu.VMEM((_C, _H), jnp.float32),
            pltpu.SemaphoreType.DMA,
            pltpu.SemaphoreType.DMA,
            pltpu.SemaphoreType.DMA,
            pltpu.SemaphoreType.DMA,
        ],
    )
    def k(src2, dst2, d_h, b_h, e_h, dhs_o, bhs_o, ehd_o,
          srcv, dstv, d0, b0, e0, d1, b1, e1, g0, g1, st0, st1):
        wid = lax.axis_index("s") * _NC + lax.axis_index("c")
        cb = wid * _CPW

        pltpu.sync_copy(src2.at[pl.ds(cb, _CPW)], srcv)
        pltpu.sync_copy(dst2.at[pl.ds(cb, _CPW)], dstv)

        def g_issue(jj, sd, sb, se, sem):
            pltpu.async_copy(d_h.at[srcv.at[jj]], sd, sem)
            pltpu.async_copy(b_h.at[srcv.at[jj]], sb, sem)
            pltpu.async_copy(e_h.at[dstv.at[jj]], se, sem)

        def g_wait(jj, sd, sb, se, sem):
            pltpu.make_async_copy(d_h.at[srcv.at[jj]], sd, sem).wait()
            pltpu.make_async_copy(b_h.at[srcv.at[jj]], sb, sem).wait()
            pltpu.make_async_copy(e_h.at[dstv.at[jj]], se, sem).wait()

        def s_issue(jj, sd, sb, se, sem):
            base = (cb + jj) * _C
            pltpu.async_copy(sd, dhs_o.at[pl.ds(base, _C)], sem)
            pltpu.async_copy(sb, bhs_o.at[pl.ds(base, _C)], sem)
            pltpu.async_copy(se, ehd_o.at[pl.ds(base, _C)], sem)

        def s_wait(jj, sd, sb, se, sem):
            base = (cb + jj) * _C
            pltpu.make_async_copy(sd, dhs_o.at[pl.ds(base, _C)], sem).wait()
            pltpu.make_async_copy(sb, bhs_o.at[pl.ds(base, _C)], sem).wait()
            pltpu.make_async_copy(se, ehd_o.at[pl.ds(base, _C)], sem).wait()

        g_issue(0, d0, b0, e0, g0)

        def body(j, carry):
            j0 = 2 * j
            j1 = 2 * j + 1

            @pl.when(j > 0)
            def _():
                s_wait(j0 - 1, d1, b1, e1, st1)

            g_issue(j1, d1, b1, e1, g1)
            g_wait(j0, d0, b0, e0, g0)
            s_issue(j0, d0, b0, e0, st0)

            @pl.when(j < _HCPW - 1)
            def _():
                s_wait(j0, d0, b0, e0, st0)
                g_issue(j0 + 2, d0, b0, e0, g0)

            g_wait(j1, d1, b1, e1, g1)
            s_issue(j1, d1, b1, e1, st1)
            return carry

        lax.fori_loop(0, _HCPW, body, 0)
        s_wait(_CPW - 2, d0, b0, e0, st0)
        s_wait(_CPW - 1, d1, b1, e1, st1)

    return k


@functools.cache
def _gather2():
    @functools.partial(
        pl.kernel,
        out_type=(_f32((_EP, _H)), _f32((_EP, _H))),
        mesh=_sc_mesh(),
        scratch_types=[
            pltpu.VMEM((_CPW, _C), jnp.int32),
            pltpu.VMEM((_CPW, _C), jnp.int32),
            pltpu.VMEM((_C, _H), jnp.float32),
            pltpu.VMEM((_C, _H), jnp.float32),
            pltpu.VMEM((_C, _H), jnp.float32),
            pltpu.VMEM((_C, _H), jnp.float32),
            pltpu.SemaphoreType.DMA,
            pltpu.SemaphoreType.DMA,
            pltpu.SemaphoreType.DMA,
            pltpu.SemaphoreType.DMA,
        ],
    )
    def k(src2, dst2, h_h, hs_o, hd_o,
          srcv, dstv, a0, b0, a1, b1, g0, g1, st0, st1):
        wid = lax.axis_index("s") * _NC + lax.axis_index("c")
        cb = wid * _CPW

        pltpu.sync_copy(src2.at[pl.ds(cb, _CPW)], srcv)
        pltpu.sync_copy(dst2.at[pl.ds(cb, _CPW)], dstv)

        def g_issue(jj, sa, sb, sem):
            pltpu.async_copy(h_h.at[srcv.at[jj]], sa, sem)
            pltpu.async_copy(h_h.at[dstv.at[jj]], sb, sem)

        def g_wait(jj, sa, sb, sem):
            pltpu.make_async_copy(h_h.at[srcv.at[jj]], sa, sem).wait()
            pltpu.make_async_copy(h_h.at[dstv.at[jj]], sb, sem).wait()

        def s_issue(jj, sa, sb, sem):
            base = (cb + jj) * _C
            pltpu.async_copy(sa, hs_o.at[pl.ds(base, _C)], sem)
            pltpu.async_copy(sb, hd_o.at[pl.ds(base, _C)], sem)

        def s_wait(jj, sa, sb, sem):
            base = (cb + jj) * _C
            pltpu.make_async_copy(sa, hs_o.at[pl.ds(base, _C)], sem).wait()
            pltpu.make_async_copy(sb, hd_o.at[pl.ds(base, _C)], sem).wait()

        g_issue(0, a0, b0, g0)

        def body(j, carry):
            j0 = 2 * j
            j1 = 2 * j + 1

            @pl.when(j > 0)
            def _():
                s_wait(j0 - 1, a1, b1, st1)

            g_issue(j1, a1, b1, g1)
            g_wait(j0, a0, b0, g0)
            s_issue(j0, a0, b0, st0)

            @pl.when(j < _HCPW - 1)
            def _():
                s_wait(j0, a0, b0, st0)
                g_issue(j0 + 2, a0, b0, g0)

            g_wait(j1, a1, b1, g1)
            s_issue(j1, a1, b1, st1)
            return carry

        lax.fori_loop(0, _HCPW, body, 0)
        s_wait(_CPW - 2, a0, b0, st0)
        s_wait(_CPW - 1, a1, b1, st1)

    return k


@functools.cache
def _scatter2():
    @functools.partial(
        pl.kernel,
        out_type=(_f32((_N, _H)), _f32((_N, _H))),
        mesh=_sc_mesh(),
        scratch_types=[
            pltpu.VMEM((_CPS, _C), jnp.int32),
            pltpu.VMEM((_C, _H), jnp.float32),
            pltpu.VMEM((_C, _H), jnp.float32),
            pltpu.VMEM_SHARED((_NP, _H), jnp.float32),
            pltpu.SemaphoreType.DMA,
            pltpu.SemaphoreType.DMA,
        ],
    )
    def k(msg_h, sig_h, dst2, zeros_h, num_o, den_o,
          idxv, v0, v1, accum, l0, l1):
        c = lax.axis_index("c")
        s = lax.axis_index("s")
        sb = s * _CPS

        pltpu.sync_copy(zeros_h, accum.at[pl.ds(s * _RPS, _RPS)])
        pltpu.sync_copy(dst2.at[pl.ds(sb, _CPS)], idxv)
        plsc.subcore_barrier()

        def run(val_h):
            def v_issue(jj, v, sem):
                base = (sb + jj) * _C
                pltpu.async_copy(val_h.at[pl.ds(base, _C)], v, sem)

            def v_wait(jj, v, sem):
                base = (sb + jj) * _C
                pltpu.make_async_copy(
                    val_h.at[pl.ds(base, _C)], v, sem).wait()

            v_issue(0, v0, l0)

            def body(j, carry):
                j0 = 2 * j
                j1 = 2 * j + 1
                v_issue(j1, v1, l1)
                v_wait(j0, v0, l0)
                pltpu.sync_copy(v0, accum.at[idxv.at[j0]], add=True)

                @pl.when(j < _HCPS - 1)
                def _():
                    v_issue(j0 + 2, v0, l0)

                v_wait(j1, v1, l1)
                pltpu.sync_copy(v1, accum.at[idxv.at[j1]], add=True)
                return carry

            lax.fori_loop(0, _HCPS, body, 0)

        @pl.when(c == 0)
        def _():
            run(msg_h)

        @pl.when(c == 1)
        def _():
            run(sig_h)

        plsc.subcore_barrier()

        last = _N - _RPS * (_NS - 1)

        @pl.when(jnp.logical_and(c == 0, s < _NS - 1))
        def _():
            pltpu.sync_copy(accum.at[pl.ds(s * _RPS, _RPS)],
                            num_o.at[pl.ds(s * _RPS, _RPS)])

        @pl.when(jnp.logical_and(c == 0, s == _NS - 1))
        def _():
            pltpu.sync_copy(accum.at[pl.ds((_NS - 1) * _RPS, last)],
                            num_o.at[pl.ds((_NS - 1) * _RPS, last)])

        @pl.when(jnp.logical_and(c == 1, s < _NS - 1))
        def _():
            pltpu.sync_copy(accum.at[pl.ds(s * _RPS, _RPS)],
                            den_o.at[pl.ds(s * _RPS, _RPS)])

        @pl.when(jnp.logical_and(c == 1, s == _NS - 1))
        def _():
            pltpu.sync_copy(accum.at[pl.ds((_NS - 1) * _RPS, last)],
                            den_o.at[pl.ds((_NS - 1) * _RPS, last)])

    return k


def _row_spec(b, w):
    return pl.BlockSpec((b, w), lambda i: (i, 0))


def _col_spec(b, w, col):
    return pl.BlockSpec((b, w), lambda i, _c=col: (i, _c))


def _full_spec(r, w):
    return pl.BlockSpec((r, w), lambda i: (0, 0))


def _h0_body(pe_r, w_r, b_r, out_r):
    out_r[...] = (jnp.dot(pe_r[...], w_r[...],
                          preferred_element_type=jnp.float32) + b_r[...])


@functools.cache
def _h0_call():
    return pl.pallas_call(
        _h0_body,
        grid=(_GN,),
        in_specs=[_row_spec(_BN, 10), _full_spec(10, _H), _full_spec(1, _H)],
        out_specs=_row_spec(_BN, _H),
        out_shape=_f32((_N, _H)),
    )


def _eh0_body(e_r, w1_r, b1_r, w2_r, b2_r, out_r):
    t = jnp.dot(e_r[...], w1_r[...], preferred_element_type=jnp.float32)
    t = jnp.maximum(t + b1_r[...], 0.0)
    out_r[...] = (jnp.dot(t, w2_r[...], preferred_element_type=jnp.float32)
                  + b2_r[...])


@functools.cache
def _eh0_call():
    return pl.pallas_call(
        _eh0_body,
        grid=(_GE,),
        in_specs=[_row_spec(_BE, 16), _full_spec(16, 16), _full_spec(1, 16),
                  _full_spec(16, _H), _full_spec(1, _H)],
        out_specs=_row_spec(_BE, _H),
        out_shape=_f32((_EP, _H)),
    )


def _node_mm_body(h_r, w_r, b_r, a_o, d_o, b_o, ew_o):
    r = jnp.dot(h_r[...], w_r[...], preferred_element_type=jnp.float32)
    r = r + b_r[...]
    a_o[...] = r[:, 0 * _H:1 * _H]
    d_o[...] = r[:, 1 * _H:2 * _H]
    b_o[...] = r[:, 2 * _H:3 * _H]
    ew_o[...] = r[:, 3 * _H:4 * _H]


@functools.cache
def _node_mm_call():
    return pl.pallas_call(
        _node_mm_body,
        grid=(_GN,),
        in_specs=[_row_spec(_BN, _H), _full_spec(_H, 4 * _H),
                  _full_spec(1, 4 * _H)],
        out_specs=tuple(_row_spec(_BN, _H) for _ in range(4)),
        out_shape=tuple(_f32((_N, _H)) for _ in range(4)),
    )


def _edge1_body(eh_r, cw_r, cb_r, dhs_r, ehd_r, enew_o, stats_o):
    i = pl.program_id(0)
    v = jnp.dot(eh_r[...], cw_r[...], preferred_element_type=jnp.float32)
    v = v + cb_r[...] + dhs_r[...] + ehd_r[...]
    enew_o[...] = v
    rows = lax.broadcasted_iota(jnp.int32, (_BE, _H), 0) + i * _BE
    vm = jnp.where(rows < _E, v, 0.0)
    st = jnp.concatenate(
        [jnp.sum(vm, axis=0, keepdims=True),
         jnp.sum(vm * vm, axis=0, keepdims=True),
         jnp.zeros((6, _H), jnp.float32)], axis=0)

    @pl.when(i == 0)
    def _():
        stats_o[...] = st

    @pl.when(i > 0)
    def _():
        stats_o[...] += st


@functools.cache
def _edge1_call():
    return pl.pallas_call(
        _edge1_body,
        grid=(_GE,),
        in_specs=[_row_spec(_BE, _H), _full_spec(_H, _H), _full_spec(1, _H),
                  _row_spec(_BE, _H), _row_spec(_BE, _H)],
        out_specs=(_row_spec(_BE, _H), _full_spec(8, _H)),
        out_shape=(_f32((_EP, _H)), _f32((8, _H))),
    )


def _edge2_body(eh_r, enew_r, bhs_r, st_r, g_r, b_r, ehn_o, sig_o, msg_o):
    st = st_r[...]
    m = st[0:1, :] * (1.0 / _E)
    var = st[1:2, :] * (1.0 / _E) - m * m
    inv = lax.rsqrt(var + 1e-5)
    xb = g_r[...] * (enew_r[...] - m) * inv + b_r[...]
    ehn = eh_r[...] + jnp.maximum(xb, 0.0)
    sig = jax.nn.sigmoid(ehn)
    ehn_o[...] = ehn
    sig_o[...] = sig
    msg_o[...] = sig * bhs_r[...]


@functools.cache
def _edge2_call():
    return pl.pallas_call(
        _edge2_body,
        grid=(_GE,),
        in_specs=[_row_spec(_BE, _H), _row_spec(_BE, _H),
                  _row_spec(_BE, _H),
                  _full_spec(8, _H), _full_spec(1, _H), _full_spec(1, _H)],
        out_specs=tuple(_row_spec(_BE, _H) for _ in range(3)),
        out_shape=tuple(_f32((_EP, _H)) for _ in range(3)),
    )


def _node1_body(ah_r, num_r, den_r, t_o, stats_o):
    i = pl.program_id(0)
    v = ah_r[...] + num_r[...] / (den_r[...] + 1e-6)
    t_o[...] = v
    st = jnp.concatenate(
        [jnp.sum(v, axis=0, keepdims=True),
         jnp.sum(v * v, axis=0, keepdims=True),
         jnp.zeros((6, _H), jnp.float32)], axis=0)

    @pl.when(i == 0)
    def _():
        stats_o[...] = st

    @pl.when(i > 0)
    def _():
        stats_o[...] += st


@functools.cache
def _node1_call():
    return pl.pallas_call(
        _node1_body,
        grid=(_GN,),
        in_specs=[_row_spec(_BN, _H)] * 3,
        out_specs=(_row_spec(_BN, _H), _full_spec(8, _H)),
        out_shape=(_f32((_N, _H)), _f32((8, _H))),
    )


def _node2_body(h_r, t_r, st_r, g_r, b_r, h_o):
    st = st_r[...]
    m = st[0:1, :] * (1.0 / _N)
    var = st[1:2, :] * (1.0 / _N) - m * m
    inv = lax.rsqrt(var + 1e-5)
    xb = g_r[...] * (t_r[...] - m) * inv + b_r[...]
    h_o[...] = h_r[...] + jnp.maximum(xb, 0.0)


@functools.cache
def _node2_call():
    return pl.pallas_call(
        _node2_body,
        grid=(_GN,),
        in_specs=[_row_spec(_BN, _H), _row_spec(_BN, _H), _full_spec(8, _H),
                  _full_spec(1, _H), _full_spec(1, _H)],
        out_specs=_row_spec(_BN, _H),
        out_shape=_f32((_N, _H)),
    )


def _final_body(hs_r, hd_r, eh_r, pa_r, pb_r, pc_r, p1b_r, p2_r, p2b_r,
                out_o):
    z = jnp.dot(hs_r[...], pa_r[...], preferred_element_type=jnp.float32)
    z = z + jnp.dot(hd_r[...], pb_r[...], preferred_element_type=jnp.float32)
    z = z + jnp.dot(eh_r[...], pc_r[...], preferred_element_type=jnp.float32)
    z = jnp.maximum(z + p1b_r[...], 0.0)
    out_o[...] = (jnp.dot(z, p2_r[...], preferred_element_type=jnp.float32)
                  + p2b_r[...])


@functools.cache
def _final_call():
    return pl.pallas_call(
        _final_body,
        grid=(_GE,),
        in_specs=[_row_spec(_BE, _H)] * 3
        + [_full_spec(_H, _H)] * 3
        + [_full_spec(1, _H), _full_spec(_H, 1), _full_spec(1, 1)],
        out_specs=_row_spec(_BE, 1),
        out_shape=_f32((_EP, 1)),
    )


def kernel(edge_index, x, e, pe, pe_W, pe_b, e1_W, e1_b, e2_W, e2_b,
           A_W, A_b, B_W, B_b, C_W, C_b, D_W, D_b, Ew_W, Ew_b,
           bn_h_g, bn_h_b, bn_e_g, bn_e_b, p1_W, p1_b, p2_W, p2_b):
    pad = _EP - _E
    src2 = jnp.concatenate(
        [edge_index[0], jnp.zeros((pad,), jnp.int32)]).reshape(_NCH, _C)
    dstg2 = jnp.concatenate(
        [edge_index[1], jnp.zeros((pad,), jnp.int32)]).reshape(_NCH, _C)
    dump = _DUMP + (jnp.arange(pad, dtype=jnp.int32) % (_NP - _N))
    dsts2 = jnp.concatenate([edge_index[1], dump]).reshape(_NCH, _C)
    e_p = jnp.concatenate([e, jnp.zeros((pad, 16), jnp.float32)], axis=0)

    h = _h0_call()(pe, pe_W, pe_b.reshape(1, _H))
    eh = _eh0_call()(e_p, e1_W, e1_b.reshape(1, 16), e2_W,
                     e2_b.reshape(1, _H))

    W4 = jnp.concatenate([A_W, D_W, B_W, Ew_W], axis=2)
    b4 = jnp.concatenate([A_b, D_b, B_b, Ew_b], axis=1)
    zeros_n = jnp.zeros((_RPS, _H), jnp.float32)

    for l in range(_L):
        Ah, Dh, Bh, Ewh = _node_mm_call()(h, W4[l], b4[l].reshape(1, 4 * _H))
        dhs, bhs, ehd = _gather3()(src2, dstg2, Dh, Bh, Ewh)
        enew, est = _edge1_call()(eh, C_W[l], C_b[l].reshape(1, _H), dhs, ehd)
        eh, sig, msg = _edge2_call()(eh, enew, bhs, est,
                                     bn_e_g[l].reshape(1, _H),
                                     bn_e_b[l].reshape(1, _H))
        num, den = _scatter2()(msg, sig, dsts2, zeros_n)
        t, nst = _node1_call()(Ah, num, den)
        h = _node2_call()(h, t, nst, bn_h_g[l].reshape(1, _H),
                          bn_h_b[l].reshape(1, _H))

    hs, hd = _gather2()(src2, dstg2, h)
    scores = _final_call()(hs, hd, eh,
                           p1_W[0 * _H:1 * _H], p1_W[1 * _H:2 * _H],
                           p1_W[2 * _H:3 * _H], p1_b.reshape(1, _H),
                           p2_W, p2_b.reshape(1, 1))
    return scores[:_E]


# speedup vs baseline: 1.4270x; 1.1026x over previous
import functools

import jax
import jax.numpy as jnp
from jax import lax
from jax.experimental import pallas as pl
from jax.experimental.pallas import tpu as pltpu
from jax.experimental.pallas import tpu_sc as plsc

_N = 10000
_E = 160000
_H = 128
_L = 4

_NC = 2
_NS = 16
_W = _NC * _NS

_C = 128
_EP = 163840
_NCH = _EP // _C
_CPW = _NCH // _W
_HCPW = _CPW // 2
_CPS = _NCH // _NS
_HCPS = _CPS // 2
_RPS = 640
_NP = _RPS * _NS
_DUMP = _N

_BE = 8192
_BN = 5000
_GE = _EP // _BE
_GN = _N // _BN


def _f32(shape):
    return jax.ShapeDtypeStruct(shape, jnp.float32)


def _sc_mesh():
    return plsc.VectorSubcoreMesh(
        core_axis_name="c", subcore_axis_name="s", num_cores=_NC,
        num_subcores=_NS)


@functools.cache
def _gather_db_e():
    @functools.partial(
        pl.kernel,
        out_type=(jax.ShapeDtypeStruct((_EP, _H), jnp.int32),
                  _f32((_EP, _H))),
        mesh=_sc_mesh(),
        scratch_types=[
            pltpu.VMEM((_CPW, _C), jnp.int32),
            pltpu.VMEM((_CPW, _C), jnp.int32),
            pltpu.VMEM((_C, _H), jnp.int32),
            pltpu.VMEM((_C, _H), jnp.int32),
            pltpu.VMEM((_C, _H), jnp.float32),
            pltpu.VMEM((_C, _H), jnp.float32),
            pltpu.SemaphoreType.DMA,
            pltpu.SemaphoreType.DMA,
            pltpu.SemaphoreType.DMA,
            pltpu.SemaphoreType.DMA,
        ],
    )
    def k(src2, dst2, db_h, e_h, dbs_o, ehd_o,
          srcv, dstv, db0, db1, e0, e1, g0, g1, st0, st1):
        wid = lax.axis_index("s") * _NC + lax.axis_index("c")
        cb = wid * _CPW

        pltpu.sync_copy(src2.at[pl.ds(cb, _CPW)], srcv)
        pltpu.sync_copy(dst2.at[pl.ds(cb, _CPW)], dstv)

        def g_issue(jj, sd, se, sem):
            pltpu.async_copy(db_h.at[srcv.at[jj]], sd, sem)
            pltpu.async_copy(e_h.at[dstv.at[jj]], se, sem)

        def g_wait(jj, sd, se, sem):
            pltpu.make_async_copy(db_h.at[srcv.at[jj]], sd, sem).wait()
            pltpu.make_async_copy(e_h.at[dstv.at[jj]], se, sem).wait()

        def s_issue(jj, sd, se, sem):
            base = (cb + jj) * _C
            pltpu.async_copy(sd, dbs_o.at[pl.ds(base, _C)], sem)
            pltpu.async_copy(se, ehd_o.at[pl.ds(base, _C)], sem)

        def s_wait(jj, sd, se, sem):
            base = (cb + jj) * _C
            pltpu.make_async_copy(sd, dbs_o.at[pl.ds(base, _C)], sem).wait()
            pltpu.make_async_copy(se, ehd_o.at[pl.ds(base, _C)], sem).wait()

        g_issue(0, db0, e0, g0)

        def body(j, carry):
            j0 = 2 * j
            j1 = 2 * j + 1

            @pl.when(j > 0)
            def _():
                s_wait(j0 - 1, db1, e1, st1)

            g_issue(j1, db1, e1, g1)
            g_wait(j0, db0, e0, g0)
            s_issue(j0, db0, e0, st0)

            @pl.when(j < _HCPW - 1)
            def _():
                s_wait(j0, db0, e0, st0)
                g_issue(j0 + 2, db0, e0, g0)

            g_wait(j1, db1, e1, g1)
            s_issue(j1, db1, e1, st1)
            return carry

        lax.fori_loop(0, _HCPW, body, 0)
        s_wait(_CPW - 2, db0, e0, st0)
        s_wait(_CPW - 1, db1, e1, st1)

    return k


@functools.cache
def _gather2():
    @functools.partial(
        pl.kernel,
        out_type=(_f32((_EP, _H)), _f32((_EP, _H))),
        mesh=_sc_mesh(),
        scratch_types=[
            pltpu.VMEM((_CPW, _C), jnp.int32),
            pltpu.VMEM((_CPW, _C), jnp.int32),
            pltpu.VMEM((_C, _H), jnp.float32),
            pltpu.VMEM((_C, _H), jnp.float32),
            pltpu.VMEM((_C, _H), jnp.float32),
            pltpu.VMEM((_C, _H), jnp.float32),
            pltpu.SemaphoreType.DMA,
            pltpu.SemaphoreType.DMA,
            pltpu.SemaphoreType.DMA,
            pltpu.SemaphoreType.DMA,
        ],
    )
    def k(src2, dst2, h_h, hs_o, hd_o,
          srcv, dstv, a0, b0, a1, b1, g0, g1, st0, st1):
        wid = lax.axis_index("s") * _NC + lax.axis_index("c")
        cb = wid * _CPW

        pltpu.sync_copy(src2.at[pl.ds(cb, _CPW)], srcv)
        pltpu.sync_copy(dst2.at[pl.ds(cb, _CPW)], dstv)

        def g_issue(jj, sa, sb, sem):
            pltpu.async_copy(h_h.at[srcv.at[jj]], sa, sem)
            pltpu.async_copy(h_h.at[dstv.at[jj]], sb, sem)

        def g_wait(jj, sa, sb, sem):
            pltpu.make_async_copy(h_h.at[srcv.at[jj]], sa, sem).wait()
            pltpu.make_async_copy(h_h.at[dstv.at[jj]], sb, sem).wait()

        def s_issue(jj, sa, sb, sem):
            base = (cb + jj) * _C
            pltpu.async_copy(sa, hs_o.at[pl.ds(base, _C)], sem)
            pltpu.async_copy(sb, hd_o.at[pl.ds(base, _C)], sem)

        def s_wait(jj, sa, sb, sem):
            base = (cb + jj) * _C
            pltpu.make_async_copy(sa, hs_o.at[pl.ds(base, _C)], sem).wait()
            pltpu.make_async_copy(sb, hd_o.at[pl.ds(base, _C)], sem).wait()

        g_issue(0, a0, b0, g0)

        def body(j, carry):
            j0 = 2 * j
            j1 = 2 * j + 1

            @pl.when(j > 0)
            def _():
                s_wait(j0 - 1, a1, b1, st1)

            g_issue(j1, a1, b1, g1)
            g_wait(j0, a0, b0, g0)
            s_issue(j0, a0, b0, st0)

            @pl.when(j < _HCPW - 1)
            def _():
                s_wait(j0, a0, b0, st0)
                g_issue(j0 + 2, a0, b0, g0)

            g_wait(j1, a1, b1, g1)
            s_issue(j1, a1, b1, st1)
            return carry

        lax.fori_loop(0, _HCPW, body, 0)
        s_wait(_CPW - 2, a0, b0, st0)
        s_wait(_CPW - 1, a1, b1, st1)

    return k


@functools.cache
def _scatter2():
    @functools.partial(
        pl.kernel,
        out_type=(_f32((_N, _H)), _f32((_N, _H))),
        mesh=_sc_mesh(),
        scratch_types=[
            pltpu.VMEM((_CPS, _C), jnp.int32),
            pltpu.VMEM((_C, _H), jnp.float32),
            pltpu.VMEM((_C, _H), jnp.float32),
            pltpu.VMEM_SHARED((_NP, _H), jnp.float32),
            pltpu.SemaphoreType.DMA,
            pltpu.SemaphoreType.DMA,
        ],
    )
    def k(msg_h, sig_h, dst2, zeros_h, num_o, den_o,
          idxv, v0, v1, accum, l0, l1):
        c = lax.axis_index("c")
        s = lax.axis_index("s")
        sb = s * _CPS

        pltpu.sync_copy(zeros_h, accum.at[pl.ds(s * _RPS, _RPS)])
        pltpu.sync_copy(dst2.at[pl.ds(sb, _CPS)], idxv)
        plsc.subcore_barrier()

        def run(val_h):
            def v_issue(jj, v, sem):
                base = (sb + jj) * _C
                pltpu.async_copy(val_h.at[pl.ds(base, _C)], v, sem)

            def v_wait(jj, v, sem):
                base = (sb + jj) * _C
                pltpu.make_async_copy(
                    val_h.at[pl.ds(base, _C)], v, sem).wait()

            v_issue(0, v0, l0)

            def body(j, carry):
                j0 = 2 * j
                j1 = 2 * j + 1
                v_issue(j1, v1, l1)
                v_wait(j0, v0, l0)
                pltpu.sync_copy(v0, accum.at[idxv.at[j0]], add=True)

                @pl.when(j < _HCPS - 1)
                def _():
                    v_issue(j0 + 2, v0, l0)

                v_wait(j1, v1, l1)
                pltpu.sync_copy(v1, accum.at[idxv.at[j1]], add=True)
                return carry

            lax.fori_loop(0, _HCPS, body, 0)

        @pl.when(c == 0)
        def _():
            run(msg_h)

        @pl.when(c == 1)
        def _():
            run(sig_h)

        plsc.subcore_barrier()

        last = _N - _RPS * (_NS - 1)

        @pl.when(jnp.logical_and(c == 0, s < _NS - 1))
        def _():
            pltpu.sync_copy(accum.at[pl.ds(s * _RPS, _RPS)],
                            num_o.at[pl.ds(s * _RPS, _RPS)])

        @pl.when(jnp.logical_and(c == 0, s == _NS - 1))
        def _():
            pltpu.sync_copy(accum.at[pl.ds((_NS - 1) * _RPS, last)],
                            num_o.at[pl.ds((_NS - 1) * _RPS, last)])

        @pl.when(jnp.logical_and(c == 1, s < _NS - 1))
        def _():
            pltpu.sync_copy(accum.at[pl.ds(s * _RPS, _RPS)],
                            den_o.at[pl.ds(s * _RPS, _RPS)])

        @pl.when(jnp.logical_and(c == 1, s == _NS - 1))
        def _():
            pltpu.sync_copy(accum.at[pl.ds((_NS - 1) * _RPS, last)],
                            den_o.at[pl.ds((_NS - 1) * _RPS, last)])

    return k


def _row_spec(b, w):
    return pl.BlockSpec((b, w), lambda i: (i, 0))


def _col_spec(b, w, col):
    return pl.BlockSpec((b, w), lambda i, _c=col: (i, _c))


def _full_spec(r, w):
    return pl.BlockSpec((r, w), lambda i: (0, 0))


def _h0_body(pe_r, w_r, b_r, out_r):
    out_r[...] = (jnp.dot(pe_r[...], w_r[...],
                          preferred_element_type=jnp.float32) + b_r[...])


@functools.cache
def _h0_call():
    return pl.pallas_call(
        _h0_body,
        grid=(_GN,),
        in_specs=[_row_spec(_BN, 10), _full_spec(10, _H), _full_spec(1, _H)],
        out_specs=_row_spec(_BN, _H),
        out_shape=_f32((_N, _H)),
    )


def _eh0_body(e_r, w1_r, b1_r, w2_r, b2_r, out_r):
    t = jnp.dot(e_r[...], w1_r[...], preferred_element_type=jnp.float32)
    t = jnp.maximum(t + b1_r[...], 0.0)
    out_r[...] = (jnp.dot(t, w2_r[...], preferred_element_type=jnp.float32)
                  + b2_r[...])


@functools.cache
def _eh0_call():
    return pl.pallas_call(
        _eh0_body,
        grid=(_GE,),
        in_specs=[_row_spec(_BE, 16), _full_spec(16, 16), _full_spec(1, 16),
                  _full_spec(16, _H), _full_spec(1, _H)],
        out_specs=_row_spec(_BE, _H),
        out_shape=_f32((_EP, _H)),
    )


def _node_mm_body(h_r, w_r, b_r, a_o, db_o, ew_o):
    r = jnp.dot(h_r[...], w_r[...], preferred_element_type=jnp.float32)
    r = r + b_r[...]
    a_o[...] = r[:, 0 * _H:1 * _H]
    du = lax.bitcast_convert_type(r[:, 1 * _H:2 * _H], jnp.uint32)
    bu = lax.bitcast_convert_type(r[:, 2 * _H:3 * _H], jnp.uint32)
    du = (du + jnp.uint32(0x8000)) >> 16
    bu = (bu + jnp.uint32(0x8000)) & jnp.uint32(0xFFFF0000)
    db_o[...] = lax.bitcast_convert_type(du | bu, jnp.int32)
    ew_o[...] = r[:, 3 * _H:4 * _H]


@functools.cache
def _node_mm_call():
    return pl.pallas_call(
        _node_mm_body,
        grid=(_GN,),
        in_specs=[_row_spec(_BN, _H), _full_spec(_H, 4 * _H),
                  _full_spec(1, 4 * _H)],
        out_specs=(_row_spec(_BN, _H), _row_spec(_BN, _H),
                   _row_spec(_BN, _H)),
        out_shape=(_f32((_N, _H)),
                   jax.ShapeDtypeStruct((_N, _H), jnp.int32),
                   _f32((_N, _H))),
    )


def _edge1_body(eh_r, cw_r, cb_r, dbw_r, ehd_r, enew_o, stats_o):
    i = pl.program_id(0)
    dhs = lax.bitcast_convert_type(
        lax.shift_left(dbw_r[...], 16), jnp.float32)
    v = jnp.dot(eh_r[...], cw_r[...], preferred_element_type=jnp.float32)
    v = v + cb_r[...] + dhs + ehd_r[...]
    enew_o[...] = v
    rows = lax.broadcasted_iota(jnp.int32, (_BE, _H), 0) + i * _BE
    vm = jnp.where(rows < _E, v, 0.0)
    st = jnp.concatenate(
        [jnp.sum(vm, axis=0, keepdims=True),
         jnp.sum(vm * vm, axis=0, keepdims=True),
         jnp.zeros((6, _H), jnp.float32)], axis=0)

    @pl.when(i == 0)
    def _():
        stats_o[...] = st

    @pl.when(i > 0)
    def _():
        stats_o[...] += st


@functools.cache
def _edge1_call():
    return pl.pallas_call(
        _edge1_body,
        grid=(_GE,),
        in_specs=[_row_spec(_BE, _H), _full_spec(_H, _H), _full_spec(1, _H),
                  _row_spec(_BE, _H), _row_spec(_BE, _H)],
        out_specs=(_row_spec(_BE, _H), _full_spec(8, _H)),
        out_shape=(_f32((_EP, _H)), _f32((8, _H))),
    )


def _edge2_body(eh_r, enew_r, dbw_r, st_r, g_r, b_r, ehn_o, sig_o, msg_o):
    bhs = lax.bitcast_convert_type(
        dbw_r[...] & jnp.int32(-65536), jnp.float32)
    st = st_r[...]
    m = st[0:1, :] * (1.0 / _E)
    var = st[1:2, :] * (1.0 / _E) - m * m
    inv = lax.rsqrt(var + 1e-5)
    xb = g_r[...] * (enew_r[...] - m) * inv + b_r[...]
    ehn = eh_r[...] + jnp.maximum(xb, 0.0)
    sig = jax.nn.sigmoid(ehn)
    ehn_o[...] = ehn
    sig_o[...] = sig
    msg_o[...] = sig * bhs


@functools.cache
def _edge2_call():
    return pl.pallas_call(
        _edge2_body,
        grid=(_GE,),
        in_specs=[_row_spec(_BE, _H), _row_spec(_BE, _H),
                  _row_spec(_BE, _H),
                  _full_spec(8, _H), _full_spec(1, _H), _full_spec(1, _H)],
        out_specs=tuple(_row_spec(_BE, _H) for _ in range(3)),
        out_shape=tuple(_f32((_EP, _H)) for _ in range(3)),
    )


def _node1_body(ah_r, num_r, den_r, t_o, stats_o):
    i = pl.program_id(0)
    v = ah_r[...] + num_r[...] / (den_r[...] + 1e-6)
    t_o[...] = v
    st = jnp.concatenate(
        [jnp.sum(v, axis=0, keepdims=True),
         jnp.sum(v * v, axis=0, keepdims=True),
         jnp.zeros((6, _H), jnp.float32)], axis=0)

    @pl.when(i == 0)
    def _():
        stats_o[...] = st

    @pl.when(i > 0)
    def _():
        stats_o[...] += st


@functools.cache
def _node1_call():
    return pl.pallas_call(
        _node1_body,
        grid=(_GN,),
        in_specs=[_row_spec(_BN, _H)] * 3,
        out_specs=(_row_spec(_BN, _H), _full_spec(8, _H)),
        out_shape=(_f32((_N, _H)), _f32((8, _H))),
    )


def _node2_body(h_r, t_r, st_r, g_r, b_r, h_o):
    st = st_r[...]
    m = st[0:1, :] * (1.0 / _N)
    var = st[1:2, :] * (1.0 / _N) - m * m
    inv = lax.rsqrt(var + 1e-5)
    xb = g_r[...] * (t_r[...] - m) * inv + b_r[...]
    h_o[...] = h_r[...] + jnp.maximum(xb, 0.0)


@functools.cache
def _node2_call():
    return pl.pallas_call(
        _node2_body,
        grid=(_GN,),
        in_specs=[_row_spec(_BN, _H), _row_spec(_BN, _H), _full_spec(8, _H),
                  _full_spec(1, _H), _full_spec(1, _H)],
        out_specs=_row_spec(_BN, _H),
        out_shape=_f32((_N, _H)),
    )


def _final_body(hs_r, hd_r, eh_r, pa_r, pb_r, pc_r, p1b_r, p2_r, p2b_r,
                out_o):
    z = jnp.dot(hs_r[...], pa_r[...], preferred_element_type=jnp.float32)
    z = z + jnp.dot(hd_r[...], pb_r[...], preferred_element_type=jnp.float32)
    z = z + jnp.dot(eh_r[...], pc_r[...], preferred_element_type=jnp.float32)
    z = jnp.maximum(z + p1b_r[...], 0.0)
    out_o[...] = (jnp.dot(z, p2_r[...], preferred_element_type=jnp.float32)
                  + p2b_r[...])


@functools.cache
def _final_call():
    return pl.pallas_call(
        _final_body,
        grid=(_GE,),
        in_specs=[_row_spec(_BE, _H)] * 3
        + [_full_spec(_H, _H)] * 3
        + [_full_spec(1, _H), _full_spec(_H, 1), _full_spec(1, 1)],
        out_specs=_row_spec(_BE, 1),
        out_shape=_f32((_EP, 1)),
    )


def kernel(edge_index, x, e, pe, pe_W, pe_b, e1_W, e1_b, e2_W, e2_b,
           A_W, A_b, B_W, B_b, C_W, C_b, D_W, D_b, Ew_W, Ew_b,
           bn_h_g, bn_h_b, bn_e_g, bn_e_b, p1_W, p1_b, p2_W, p2_b):
    pad = _EP - _E
    src2 = jnp.concatenate(
        [edge_index[0], jnp.zeros((pad,), jnp.int32)]).reshape(_NCH, _C)
    dstg2 = jnp.concatenate(
        [edge_index[1], jnp.zeros((pad,), jnp.int32)]).reshape(_NCH, _C)
    dump = _DUMP + (jnp.arange(pad, dtype=jnp.int32) % (_NP - _N))
    dsts2 = jnp.concatenate([edge_index[1], dump]).reshape(_NCH, _C)
    e_p = jnp.concatenate([e, jnp.zeros((pad, 16), jnp.float32)], axis=0)

    h = _h0_call()(pe, pe_W, pe_b.reshape(1, _H))
    eh = _eh0_call()(e_p, e1_W, e1_b.reshape(1, 16), e2_W,
                     e2_b.reshape(1, _H))

    W4 = jnp.concatenate([A_W, D_W, B_W, Ew_W], axis=2)
    b4 = jnp.concatenate([A_b, D_b, B_b, Ew_b], axis=1)
    zeros_n = jnp.zeros((_RPS, _H), jnp.float32)

    for l in range(_L):
        Ah, DBw, Ewh = _node_mm_call()(h, W4[l], b4[l].reshape(1, 4 * _H))
        dbs, ehd = _gather_db_e()(src2, dstg2, DBw, Ewh)
        enew, est = _edge1_call()(eh, C_W[l], C_b[l].reshape(1, _H), dbs, ehd)
        eh, sig, msg = _edge2_call()(eh, enew, dbs, est,
                                     bn_e_g[l].reshape(1, _H),
                                     bn_e_b[l].reshape(1, _H))
        num, den = _scatter2()(msg, sig, dsts2, zeros_n)
        t, nst = _node1_call()(Ah, num, den)
        h = _node2_call()(h, t, nst, bn_h_g[l].reshape(1, _H),
                          bn_h_b[l].reshape(1, _H))

    hs, hd = _gather2()(src2, dstg2, h)
    scores = _final_call()(hs, hd, eh,
                           p1_W[0 * _H:1 * _H], p1_W[1 * _H:2 * _H],
                           p1_W[2 * _H:3 * _H], p1_b.reshape(1, _H),
                           p2_W, p2_b.reshape(1, 1))
    return scores[:_E]


# speedup vs baseline: 1.4272x; 1.0001x over previous
import functools

import jax
import jax.numpy as jnp
from jax import lax
from jax.experimental import pallas as pl
from jax.experimental.pallas import tpu as pltpu
from jax.experimental.pallas import tpu_sc as plsc

_N = 10000
_E = 160000
_H = 128
_L = 4

_NC = 2
_NS = 16
_W = _NC * _NS

_C = 128
_EP = 163840
_NCH = _EP // _C
_CPW = _NCH // _W
_HCPW = _CPW // 2
_CPS = _NCH // _NS
_HCPS = _CPS // 2
_RPS = 640
_NP = _RPS * _NS
_DUMP = _N

_BE = 8192
_BN = 5000
_GE = _EP // _BE
_GN = _N // _BN


def _f32(shape):
    return jax.ShapeDtypeStruct(shape, jnp.float32)


def _sc_mesh():
    return plsc.VectorSubcoreMesh(
        core_axis_name="c", subcore_axis_name="s", num_cores=_NC,
        num_subcores=_NS)


@functools.cache
def _gather_db_e():
    @functools.partial(
        pl.kernel,
        out_type=(jax.ShapeDtypeStruct((_EP, _H), jnp.int32),
                  _f32((_EP, _H))),
        mesh=_sc_mesh(),
        scratch_types=[
            pltpu.VMEM((_CPW, _C), jnp.int32),
            pltpu.VMEM((_CPW, _C), jnp.int32),
            pltpu.VMEM((_C, _H), jnp.int32),
            pltpu.VMEM((_C, _H), jnp.int32),
            pltpu.VMEM((_C, _H), jnp.float32),
            pltpu.VMEM((_C, _H), jnp.float32),
            pltpu.SemaphoreType.DMA,
            pltpu.SemaphoreType.DMA,
            pltpu.SemaphoreType.DMA,
            pltpu.SemaphoreType.DMA,
        ],
    )
    def k(src2, dst2, db_h, e_h, dbs_o, ehd_o,
          srcv, dstv, db0, db1, e0, e1, g0, g1, st0, st1):
        wid = lax.axis_index("s") * _NC + lax.axis_index("c")
        cb = wid * _CPW

        pltpu.sync_copy(src2.at[pl.ds(cb, _CPW)], srcv)
        pltpu.sync_copy(dst2.at[pl.ds(cb, _CPW)], dstv)

        def g_issue(jj, sd, se, sem):
            pltpu.async_copy(db_h.at[srcv.at[jj]], sd, sem)
            pltpu.async_copy(e_h.at[dstv.at[jj]], se, sem)

        def g_wait(jj, sd, se, sem):
            pltpu.make_async_copy(db_h.at[srcv.at[jj]], sd, sem).wait()
            pltpu.make_async_copy(e_h.at[dstv.at[jj]], se, sem).wait()

        def s_issue(jj, sd, se, sem):
            base = (cb + jj) * _C
            pltpu.async_copy(sd, dbs_o.at[pl.ds(base, _C)], sem)
            pltpu.async_copy(se, ehd_o.at[pl.ds(base, _C)], sem)

        def s_wait(jj, sd, se, sem):
            base = (cb + jj) * _C
            pltpu.make_async_copy(sd, dbs_o.at[pl.ds(base, _C)], sem).wait()
            pltpu.make_async_copy(se, ehd_o.at[pl.ds(base, _C)], sem).wait()

        g_issue(0, db0, e0, g0)

        def body(j, carry):
            j0 = 2 * j
            j1 = 2 * j + 1

            @pl.when(j > 0)
            def _():
                s_wait(j0 - 1, db1, e1, st1)

            g_issue(j1, db1, e1, g1)
            g_wait(j0, db0, e0, g0)
            s_issue(j0, db0, e0, st0)

            @pl.when(j < _HCPW - 1)
            def _():
                s_wait(j0, db0, e0, st0)
                g_issue(j0 + 2, db0, e0, g0)

            g_wait(j1, db1, e1, g1)
            s_issue(j1, db1, e1, st1)
            return carry

        lax.fori_loop(0, _HCPW, body, 0)
        s_wait(_CPW - 2, db0, e0, st0)
        s_wait(_CPW - 1, db1, e1, st1)

    return k


@functools.cache
def _gather2():
    @functools.partial(
        pl.kernel,
        out_type=(_f32((_EP, _H)), _f32((_EP, _H))),
        mesh=_sc_mesh(),
        scratch_types=[
            pltpu.VMEM((_CPW, _C), jnp.int32),
            pltpu.VMEM((_CPW, _C), jnp.int32),
            pltpu.VMEM((_C, _H), jnp.float32),
            pltpu.VMEM((_C, _H), jnp.float32),
            pltpu.VMEM((_C, _H), jnp.float32),
            pltpu.VMEM((_C, _H), jnp.float32),
            pltpu.SemaphoreType.DMA,
            pltpu.SemaphoreType.DMA,
            pltpu.SemaphoreType.DMA,
            pltpu.SemaphoreType.DMA,
        ],
    )
    def k(src2, dst2, h_h, hs_o, hd_o,
          srcv, dstv, a0, b0, a1, b1, g0, g1, st0, st1):
        wid = lax.axis_index("s") * _NC + lax.axis_index("c")
        cb = wid * _CPW

        pltpu.sync_copy(src2.at[pl.ds(cb, _CPW)], srcv)
        pltpu.sync_copy(dst2.at[pl.ds(cb, _CPW)], dstv)

        def g_issue(jj, sa, sb, sem):
            pltpu.async_copy(h_h.at[srcv.at[jj]], sa, sem)
            pltpu.async_copy(h_h.at[dstv.at[jj]], sb, sem)

        def g_wait(jj, sa, sb, sem):
            pltpu.make_async_copy(h_h.at[srcv.at[jj]], sa, sem).wait()
            pltpu.make_async_copy(h_h.at[dstv.at[jj]], sb, sem).wait()

        def s_issue(jj, sa, sb, sem):
            base = (cb + jj) * _C
            pltpu.async_copy(sa, hs_o.at[pl.ds(base, _C)], sem)
            pltpu.async_copy(sb, hd_o.at[pl.ds(base, _C)], sem)

        def s_wait(jj, sa, sb, sem):
            base = (cb + jj) * _C
            pltpu.make_async_copy(sa, hs_o.at[pl.ds(base, _C)], sem).wait()
            pltpu.make_async_copy(sb, hd_o.at[pl.ds(base, _C)], sem).wait()

        g_issue(0, a0, b0, g0)

        def body(j, carry):
            j0 = 2 * j
            j1 = 2 * j + 1

            @pl.when(j > 0)
            def _():
                s_wait(j0 - 1, a1, b1, st1)

            g_issue(j1, a1, b1, g1)
            g_wait(j0, a0, b0, g0)
            s_issue(j0, a0, b0, st0)

            @pl.when(j < _HCPW - 1)
            def _():
                s_wait(j0, a0, b0, st0)
                g_issue(j0 + 2, a0, b0, g0)

            g_wait(j1, a1, b1, g1)
            s_issue(j1, a1, b1, st1)
            return carry

        lax.fori_loop(0, _HCPW, body, 0)
        s_wait(_CPW - 2, a0, b0, st0)
        s_wait(_CPW - 1, a1, b1, st1)

    return k


@functools.cache
def _scatter2():
    @functools.partial(
        pl.kernel,
        out_type=(_f32((_N, _H)), _f32((_N, _H))),
        mesh=_sc_mesh(),
        scratch_types=[
            pltpu.VMEM((_CPS, _C), jnp.int32),
            pltpu.VMEM((_C, _H), jnp.float32),
            pltpu.VMEM((_C, _H), jnp.float32),
            pltpu.VMEM_SHARED((_NP, _H), jnp.float32),
            pltpu.SemaphoreType.DMA,
            pltpu.SemaphoreType.DMA,
        ],
    )
    def k(msg_h, sig_h, dst2, zeros_h, num_o, den_o,
          idxv, v0, v1, accum, l0, l1):
        c = lax.axis_index("c")
        s = lax.axis_index("s")
        sb = s * _CPS

        pltpu.sync_copy(zeros_h, accum.at[pl.ds(s * _RPS, _RPS)])
        pltpu.sync_copy(dst2.at[pl.ds(sb, _CPS)], idxv)
        plsc.subcore_barrier()

        def run(val_h):
            def v_issue(jj, v, sem):
                base = (sb + jj) * _C
                pltpu.async_copy(val_h.at[pl.ds(base, _C)], v, sem)

            def v_wait(jj, v, sem):
                base = (sb + jj) * _C
                pltpu.make_async_copy(
                    val_h.at[pl.ds(base, _C)], v, sem).wait()

            v_issue(0, v0, l0)

            def body(j, carry):
                j0 = 2 * j
                j1 = 2 * j + 1
                v_issue(j1, v1, l1)
                v_wait(j0, v0, l0)
                pltpu.sync_copy(v0, accum.at[idxv.at[j0]], add=True)

                @pl.when(j < _HCPS - 1)
                def _():
                    v_issue(j0 + 2, v0, l0)

                v_wait(j1, v1, l1)
                pltpu.sync_copy(v1, accum.at[idxv.at[j1]], add=True)
                return carry

            lax.fori_loop(0, _HCPS, body, 0)

        @pl.when(c == 0)
        def _():
            run(msg_h)

        @pl.when(c == 1)
        def _():
            run(sig_h)

        plsc.subcore_barrier()

        last = _N - _RPS * (_NS - 1)

        @pl.when(jnp.logical_and(c == 0, s < _NS - 1))
        def _():
            pltpu.sync_copy(accum.at[pl.ds(s * _RPS, _RPS)],
                            num_o.at[pl.ds(s * _RPS, _RPS)])

        @pl.when(jnp.logical_and(c == 0, s == _NS - 1))
        def _():
            pltpu.sync_copy(accum.at[pl.ds((_NS - 1) * _RPS, last)],
                            num_o.at[pl.ds((_NS - 1) * _RPS, last)])

        @pl.when(jnp.logical_and(c == 1, s < _NS - 1))
        def _():
            pltpu.sync_copy(accum.at[pl.ds(s * _RPS, _RPS)],
                            den_o.at[pl.ds(s * _RPS, _RPS)])

        @pl.when(jnp.logical_and(c == 1, s == _NS - 1))
        def _():
            pltpu.sync_copy(accum.at[pl.ds((_NS - 1) * _RPS, last)],
                            den_o.at[pl.ds((_NS - 1) * _RPS, last)])

    return k


def _row_spec(b, w):
    return pl.BlockSpec((b, w), lambda i: (i, 0))


def _col_spec(b, w, col):
    return pl.BlockSpec((b, w), lambda i, _c=col: (i, _c))


def _full_spec(r, w):
    return pl.BlockSpec((r, w), lambda i: (0, 0))


def _h0_body(pe_r, w_r, b_r, out_r):
    out_r[...] = (jnp.dot(pe_r[...], w_r[...],
                          preferred_element_type=jnp.float32) + b_r[...])


@functools.cache
def _h0_call():
    return pl.pallas_call(
        _h0_body,
        grid=(_GN,),
        in_specs=[_row_spec(_BN, 10), _full_spec(10, _H), _full_spec(1, _H)],
        out_specs=_row_spec(_BN, _H),
        out_shape=_f32((_N, _H)),
    )


def _eh0_body(e_r, w1_r, b1_r, w2_r, b2_r, out_r):
    t = jnp.dot(e_r[...], w1_r[...], preferred_element_type=jnp.float32)
    t = jnp.maximum(t + b1_r[...], 0.0)
    out_r[...] = (jnp.dot(t, w2_r[...], preferred_element_type=jnp.float32)
                  + b2_r[...])


@functools.cache
def _eh0_call():
    return pl.pallas_call(
        _eh0_body,
        grid=(_GE,),
        in_specs=[_row_spec(_BE, 16), _full_spec(16, 16), _full_spec(1, 16),
                  _full_spec(16, _H), _full_spec(1, _H)],
        out_specs=_row_spec(_BE, _H),
        out_shape=_f32((_EP, _H)),
    )


def _pack2(lo, hi):
    lu = lax.bitcast_convert_type(lo, jnp.uint32)
    hu = lax.bitcast_convert_type(hi, jnp.uint32)
    lu = (lu + jnp.uint32(0x8000)) >> 16
    hu = (hu + jnp.uint32(0x8000)) & jnp.uint32(0xFFFF0000)
    return lax.bitcast_convert_type(lu | hu, jnp.int32)


def _unpack_lo(w):
    return lax.bitcast_convert_type(lax.shift_left(w, 16), jnp.float32)


def _unpack_hi(w):
    return lax.bitcast_convert_type(w & jnp.int32(-65536), jnp.float32)


def _node_mm_body(h_r, w_r, b_r, a_o, db_o, ew_o):
    r = jnp.dot(h_r[...], w_r[...], preferred_element_type=jnp.float32)
    r = r + b_r[...]
    a_o[...] = r[:, 0 * _H:1 * _H]
    db_o[...] = _pack2(r[:, 1 * _H:2 * _H], r[:, 2 * _H:3 * _H])
    ew_o[...] = r[:, 3 * _H:4 * _H]


@functools.cache
def _node_mm_call():
    return pl.pallas_call(
        _node_mm_body,
        grid=(_GN,),
        in_specs=[_row_spec(_BN, _H), _full_spec(_H, 4 * _H),
                  _full_spec(1, 4 * _H)],
        out_specs=(_row_spec(_BN, _H), _row_spec(_BN, _H),
                   _row_spec(_BN, _H)),
        out_shape=(_f32((_N, _H)),
                   jax.ShapeDtypeStruct((_N, _H), jnp.int32),
                   _f32((_N, _H))),
    )


def _edge1_body(eh_r, cw_r, cb_r, dbw_r, ehd_r, enew_o, stats_o):
    i = pl.program_id(0)
    dhs = _unpack_lo(dbw_r[...])
    v = jnp.dot(eh_r[...], cw_r[...], preferred_element_type=jnp.float32)
    v = v + cb_r[...] + dhs + ehd_r[...]
    enew_o[...] = v
    rows = lax.broadcasted_iota(jnp.int32, (_BE, _H), 0) + i * _BE
    vm = jnp.where(rows < _E, v, 0.0)
    st = jnp.concatenate(
        [jnp.sum(vm, axis=0, keepdims=True),
         jnp.sum(vm * vm, axis=0, keepdims=True),
         jnp.zeros((6, _H), jnp.float32)], axis=0)

    @pl.when(i == 0)
    def _():
        stats_o[...] = st

    @pl.when(i > 0)
    def _():
        stats_o[...] += st


@functools.cache
def _edge1_call():
    return pl.pallas_call(
        _edge1_body,
        grid=(_GE,),
        in_specs=[_row_spec(_BE, _H), _full_spec(_H, _H), _full_spec(1, _H),
                  _row_spec(_BE, _H), _row_spec(_BE, _H)],
        out_specs=(_row_spec(_BE, _H), _full_spec(8, _H)),
        out_shape=(_f32((_EP, _H)), _f32((8, _H))),
    )


def _edge2_body(eh_r, enew_r, dbw_r, st_r, g_r, b_r, ehn_o, sig_o, msg_o):
    bhs = _unpack_hi(dbw_r[...])
    st = st_r[...]
    m = st[0:1, :] * (1.0 / _E)
    var = st[1:2, :] * (1.0 / _E) - m * m
    inv = lax.rsqrt(var + 1e-5)
    xb = g_r[...] * (enew_r[...] - m) * inv + b_r[...]
    ehn = eh_r[...] + jnp.maximum(xb, 0.0)
    sig = jax.nn.sigmoid(ehn)
    ehn_o[...] = ehn
    sig_o[...] = sig
    msg_o[...] = sig * bhs


@functools.cache
def _edge2_call():
    return pl.pallas_call(
        _edge2_body,
        grid=(_GE,),
        in_specs=[_row_spec(_BE, _H), _row_spec(_BE, _H),
                  _row_spec(_BE, _H),
                  _full_spec(8, _H), _full_spec(1, _H), _full_spec(1, _H)],
        out_specs=tuple(_row_spec(_BE, _H) for _ in range(3)),
        out_shape=tuple(_f32((_EP, _H)) for _ in range(3)),
    )


def _node1_body(ah_r, num_r, den_r, t_o, stats_o):
    i = pl.program_id(0)
    v = ah_r[...] + num_r[...] / (den_r[...] + 1e-6)
    t_o[...] = v
    st = jnp.concatenate(
        [jnp.sum(v, axis=0, keepdims=True),
         jnp.sum(v * v, axis=0, keepdims=True),
         jnp.zeros((6, _H), jnp.float32)], axis=0)

    @pl.when(i == 0)
    def _():
        stats_o[...] = st

    @pl.when(i > 0)
    def _():
        stats_o[...] += st


@functools.cache
def _node1_call():
    return pl.pallas_call(
        _node1_body,
        grid=(_GN,),
        in_specs=[_row_spec(_BN, _H)] * 3,
        out_specs=(_row_spec(_BN, _H), _full_spec(8, _H)),
        out_shape=(_f32((_N, _H)), _f32((8, _H))),
    )


def _node2_body(h_r, t_r, st_r, g_r, b_r, h_o):
    st = st_r[...]
    m = st[0:1, :] * (1.0 / _N)
    var = st[1:2, :] * (1.0 / _N) - m * m
    inv = lax.rsqrt(var + 1e-5)
    xb = g_r[...] * (t_r[...] - m) * inv + b_r[...]
    h_o[...] = h_r[...] + jnp.maximum(xb, 0.0)


@functools.cache
def _node2_call():
    return pl.pallas_call(
        _node2_body,
        grid=(_GN,),
        in_specs=[_row_spec(_BN, _H), _row_spec(_BN, _H), _full_spec(8, _H),
                  _full_spec(1, _H), _full_spec(1, _H)],
        out_specs=_row_spec(_BN, _H),
        out_shape=_f32((_N, _H)),
    )


def _final_body(hs_r, hd_r, eh_r, pa_r, pb_r, pc_r, p1b_r, p2_r, p2b_r,
                out_o):
    z = jnp.dot(hs_r[...], pa_r[...], preferred_element_type=jnp.float32)
    z = z + jnp.dot(hd_r[...], pb_r[...], preferred_element_type=jnp.float32)
    z = z + jnp.dot(eh_r[...], pc_r[...], preferred_element_type=jnp.float32)
    z = jnp.maximum(z + p1b_r[...], 0.0)
    out_o[...] = (jnp.dot(z, p2_r[...], preferred_element_type=jnp.float32)
                  + p2b_r[...])


@functools.cache
def _final_call():
    return pl.pallas_call(
        _final_body,
        grid=(_GE,),
        in_specs=[_row_spec(_BE, _H)] * 3
        + [_full_spec(_H, _H)] * 3
        + [_full_spec(1, _H), _full_spec(_H, 1), _full_spec(1, 1)],
        out_specs=_row_spec(_BE, 1),
        out_shape=_f32((_EP, 1)),
    )


def kernel(edge_index, x, e, pe, pe_W, pe_b, e1_W, e1_b, e2_W, e2_b,
           A_W, A_b, B_W, B_b, C_W, C_b, D_W, D_b, Ew_W, Ew_b,
           bn_h_g, bn_h_b, bn_e_g, bn_e_b, p1_W, p1_b, p2_W, p2_b):
    pad = _EP - _E
    src2 = jnp.concatenate(
        [edge_index[0], jnp.zeros((pad,), jnp.int32)]).reshape(_NCH, _C)
    dstg2 = jnp.concatenate(
        [edge_index[1], jnp.zeros((pad,), jnp.int32)]).reshape(_NCH, _C)
    dump = _DUMP + (jnp.arange(pad, dtype=jnp.int32) % (_NP - _N))
    dsts2 = jnp.concatenate([edge_index[1], dump]).reshape(_NCH, _C)
    e_p = jnp.concatenate([e, jnp.zeros((pad, 16), jnp.float32)], axis=0)

    h = _h0_call()(pe, pe_W, pe_b.reshape(1, _H))
    eh = _eh0_call()(e_p, e1_W, e1_b.reshape(1, 16), e2_W,
                     e2_b.reshape(1, _H))

    W4 = jnp.concatenate([A_W, D_W, B_W, Ew_W], axis=2)
    b4 = jnp.concatenate([A_b, D_b, B_b, Ew_b], axis=1)
    zeros_n = jnp.zeros((_RPS, _H), jnp.float32)

    for l in range(_L):
        Ah, DBw, Ewh = _node_mm_call()(h, W4[l], b4[l].reshape(1, 4 * _H))
        dbs, ehd = _gather_db_e()(src2, dstg2, DBw, Ewh)
        enew, est = _edge1_call()(eh, C_W[l], C_b[l].reshape(1, _H), dbs, ehd)
        eh, sig, msg = _edge2_call()(eh, enew, dbs, est,
                                     bn_e_g[l].reshape(1, _H),
                                     bn_e_b[l].reshape(1, _H))
        num, den = _scatter2()(msg, sig, dsts2, zeros_n)
        t, nst = _node1_call()(Ah, num, den)
        h = _node2_call()(h, t, nst, bn_h_g[l].reshape(1, _H),
                          bn_h_b[l].reshape(1, _H))

    hs, hd = _gather2()(src2, dstg2, h)
    scores = _final_call()(hs, hd, eh,
                           p1_W[0 * _H:1 * _H], p1_W[1 * _H:2 * _H],
                           p1_W[2 * _H:3 * _H], p1_b.reshape(1, _H),
                           p2_W, p2_b.reshape(1, 1))
    return scores[:_E]
